# Initial kernel scaffold; baseline (speedup 1.0000x reference)
import jax
import jax.numpy as jnp
from jax.experimental import pallas as pl


def kernel(x, edge_index, edge_attr, batch, Wih_s, Whh_s, bih_s, bhh_s, Watt_s, Wfc_s, bfc_s, Wih_n, Whh_n, bih_n, bhh_n, Watt_n, Wfc_n, bfc_n, Wg1, bg1, Wg2, bg2, Wls, bls):
    raise NotImplementedError("write your pallas kernel here")


# speedup vs baseline: 17.9187x; 17.9187x over previous
import functools

import jax
import jax.numpy as jnp
from jax import lax
from jax.experimental import pallas as pl
from jax.experimental.pallas import tpu as pltpu
from jax.experimental.pallas import tpu_sc as plsc

N = 50001
E = 1600032
NG = 16667
LAG = 30
HID = 32
OUT = 32

NC = 2
NS = 16
NW = NC * NS

NPAD = 50176
GPAD = 16896
PPAD = 53248
E_PAD = 1638400
EPT = E_PAD // NW

R_LSTM = 512
S1_CH = 1024
S2_CH = 512
NSTR = NPAD // NS
GSTR = GPAD // NS

@functools.cache
def _mesh():
    return plsc.VectorSubcoreMesh(
        core_axis_name="c", subcore_axis_name="s",
        num_cores=NC, num_subcores=NS)


def _attlstm_body(x_ref, wih_ref, whh_ref, b_ref, watt_ref, wfc_ref,
                  bfc_ref, out_ref, hs_ref):
    R = x_ref.shape[2]
    D = x_ref.shape[1]
    whh = whh_ref[...]
    b = b_ref[...]
    hT = jnp.zeros((HID, R), jnp.float32)
    cT = jnp.zeros((HID, R), jnp.float32)
    scores = []
    for t in range(LAG):
        xtT = x_ref[t]
        if D == 1:
            gx = wih_ref[...] * xtT
        else:
            gx = jnp.dot(wih_ref[...], xtT,
                         preferred_element_type=jnp.float32)
        g = gx + jnp.dot(whh, hT, preferred_element_type=jnp.float32) + b
        i = jax.nn.sigmoid(g[0:HID])
        f = jax.nn.sigmoid(g[HID:2 * HID])
        gg = jnp.tanh(g[2 * HID:3 * HID])
        o = jax.nn.sigmoid(g[3 * HID:4 * HID])
        cT = f * cT + i * gg
        hT = o * jnp.tanh(cT)
        hs_ref[t] = hT
        scores.append(jnp.dot(watt_ref[t:t + 1, :], hT,
                              preferred_element_type=jnp.float32))
    s = jnp.concatenate(scores, axis=0)
    m = jnp.max(s, axis=0, keepdims=True)
    e = jnp.exp(s - m)
    a = e / jnp.sum(e, axis=0, keepdims=True)
    attT = jnp.zeros((HID, R), jnp.float32)
    for t in range(LAG):
        attT = attT + a[t:t + 1, :] * hs_ref[t]
    outT = (jnp.dot(wfc_ref[...], attT, preferred_element_type=jnp.float32)
            + bfc_ref[...])
    out_ref[...] = jnp.maximum(outT, 0.0)


def _attlstm(xpT, wih, whh, b2, watt, wfc, bfc2):
    _, D, B = xpT.shape
    grid = B // R_LSTM
    outT = pl.pallas_call(
        _attlstm_body,
        grid=(grid,),
        in_specs=[
            pl.BlockSpec((LAG, D, R_LSTM), lambda i: (0, 0, i)),
            pl.BlockSpec((4 * HID, D), lambda i: (0, 0)),
            pl.BlockSpec((4 * HID, HID), lambda i: (0, 0)),
            pl.BlockSpec((4 * HID, 1), lambda i: (0, 0)),
            pl.BlockSpec((LAG, HID), lambda i: (0, 0)),
            pl.BlockSpec((OUT, HID), lambda i: (0, 0)),
            pl.BlockSpec((OUT, 1), lambda i: (0, 0)),
        ],
        out_specs=pl.BlockSpec((OUT, R_LSTM), lambda i: (0, i)),
        out_shape=jax.ShapeDtypeStruct((OUT, B), jnp.float32),
        scratch_shapes=[pltpu.VMEM((LAG, HID, R_LSTM), jnp.float32)],
    )(xpT, wih, whh, b2, watt, wfc, bfc2)
    return outT.T


_RG = 3584


def _g0_body(degp_ref, emb_ref, wg1_ref, dis_ref, h1s_ref):
    deg = degp_ref[:, 0:1] + degp_ref[:, 1:2] + 1.0
    dis = lax.rsqrt(deg)
    h1 = jnp.dot(emb_ref[...], wg1_ref[...], preferred_element_type=jnp.float32)
    dis_ref[...] = dis
    h1s_ref[...] = dis * h1


def _g0(degpT, emb, wg1):
    grid = NPAD // _RG
    return pl.pallas_call(
        _g0_body,
        grid=(grid,),
        in_specs=[
            pl.BlockSpec((_RG, NC), lambda i: (i, 0)),
            pl.BlockSpec((_RG, HID), lambda i: (i, 0)),
            pl.BlockSpec((HID, HID), lambda i: (0, 0)),
        ],
        out_specs=[
            pl.BlockSpec((_RG, 1), lambda i: (i, 0)),
            pl.BlockSpec((_RG, HID), lambda i: (i, 0)),
        ],
        out_shape=[
            jax.ShapeDtypeStruct((NPAD, 1), jnp.float32),
            jax.ShapeDtypeStruct((NPAD, HID), jnp.float32),
        ],
    )(degpT, emb, wg1)


def _g1_body(aggp_ref, dis_ref, h1s_ref, bg1_ref, wg2_ref, h2s_ref):
    dis = dis_ref[...]
    x2a = dis * (aggp_ref[0] + aggp_ref[1] + h1s_ref[...]) + bg1_ref[...]
    h2 = jnp.dot(x2a, wg2_ref[...], preferred_element_type=jnp.float32)
    h2s_ref[...] = dis * h2


def _g1(aggp, dis, h1s, bg1, wg2):
    grid = NPAD // _RG
    return pl.pallas_call(
        _g1_body,
        grid=(grid,),
        in_specs=[
            pl.BlockSpec((NC, _RG, HID), lambda i: (0, i, 0)),
            pl.BlockSpec((_RG, 1), lambda i: (i, 0)),
            pl.BlockSpec((_RG, HID), lambda i: (i, 0)),
            pl.BlockSpec((1, HID), lambda i: (0, 0)),
            pl.BlockSpec((HID, OUT), lambda i: (0, 0)),
        ],
        out_specs=pl.BlockSpec((_RG, OUT), lambda i: (i, 0)),
        out_shape=jax.ShapeDtypeStruct((NPAD, OUT), jnp.float32),
    )(aggp, dis, h1s, bg1, wg2)


def _g2_body(aggp_ref, dis_ref, h2s_ref, bg2_ref, x2_ref):
    dis = dis_ref[...]
    x2_ref[...] = dis * (aggp_ref[0] + aggp_ref[1] + h2s_ref[...]) + bg2_ref[...]


def _g2(aggp, dis, h2s, bg2):
    grid = NPAD // _RG
    return pl.pallas_call(
        _g2_body,
        grid=(grid,),
        in_specs=[
            pl.BlockSpec((NC, _RG, OUT), lambda i: (0, i, 0)),
            pl.BlockSpec((_RG, 1), lambda i: (i, 0)),
            pl.BlockSpec((_RG, OUT), lambda i: (i, 0)),
            pl.BlockSpec((1, OUT), lambda i: (0, 0)),
        ],
        out_specs=pl.BlockSpec((_RG, OUT), lambda i: (i, 0)),
        out_shape=jax.ShapeDtypeStruct((NPAD, OUT), jnp.float32),
    )(aggp, dis, h2s, bg2)


_RG3 = 2112


def _g3_body(ssump_ref, cntp_ref, x1s_ref, wa_ref, wb_ref, bls_ref, out_ref):
    cnt = jnp.maximum(cntp_ref[:, 0:1] + cntp_ref[:, 1:2], 1.0)
    x2n = (ssump_ref[0] + ssump_ref[1]) / cnt
    pred = (jnp.dot(x1s_ref[...], wa_ref[...], preferred_element_type=jnp.float32)
            + jnp.dot(x2n, wb_ref[...], preferred_element_type=jnp.float32)
            + bls_ref[...])
    out_ref[...] = jnp.maximum(pred, 0.0)


def _g3(ssump, cntpT, x1s, wa, wb, bls2):
    grid = GPAD // _RG3
    return pl.pallas_call(
        _g3_body,
        grid=(grid,),
        in_specs=[
            pl.BlockSpec((NC, _RG3, OUT), lambda i: (0, i, 0)),
            pl.BlockSpec((_RG3, NC), lambda i: (i, 0)),
            pl.BlockSpec((_RG3, OUT), lambda i: (i, 0)),
            pl.BlockSpec((OUT, 1), lambda i: (0, 0)),
            pl.BlockSpec((OUT, 1), lambda i: (0, 0)),
            pl.BlockSpec((1, 1), lambda i: (0, 0)),
        ],
        out_specs=pl.BlockSpec((_RG3, 1), lambda i: (i, 0)),
        out_shape=jax.ShapeDtypeStruct((GPAD, 1), jnp.float32),
    )(ssump, cntpT, x1s, wa, wb, bls2)


def _memset(ref, n, val):
    def step(i, carry):
        ref[pl.ds(i * 16, 16)] = jnp.full((16,), val, jnp.float32)
        return carry
    lax.fori_loop(0, n // 16, step, 0)


def _s1_body(dst_ref, w_ref, batch_ref, degout_ref, cntout_ref,
             dstv, wv, batchv, zbuf, onesv, dacc, cacc):
    core = lax.axis_index("c")
    sub = lax.axis_index("s")
    wid = sub * NC + core
    _memset(zbuf, NSTR, 0.0)
    _memset(onesv, 128, 1.0)
    pltpu.sync_copy(zbuf.at[pl.ds(0, NSTR)], dacc.at[pl.ds(sub * NSTR, NSTR)])
    pltpu.sync_copy(zbuf.at[pl.ds(0, GSTR)], cacc.at[pl.ds(sub * GSTR, GSTR)])
    plsc.subcore_barrier()

    def echunk(ci, carry):
        row0 = wid * (EPT // 128) + ci * (S1_CH // 128)
        pltpu.sync_copy(dst_ref.at[pl.ds(row0, S1_CH // 128)], dstv)
        pltpu.sync_copy(w_ref.at[pl.ds(row0, S1_CH // 128)], wv)
        for j in range(S1_CH // 128):
            pltpu.sync_copy(wv.at[j], dacc.at[dstv.at[j]], add=True)
        return carry
    lax.fori_loop(0, EPT // S1_CH, echunk, 0)

    for j in range(PPAD // NW // 128):
        pltpu.sync_copy(batch_ref.at[pl.ds(wid * (PPAD // NW) + j * 128, 128)],
                        batchv.at[j])
    for j in range(PPAD // NW // 128):
        pltpu.sync_copy(onesv, cacc.at[batchv.at[j]], add=True)
    plsc.subcore_barrier()

    pltpu.sync_copy(dacc.at[pl.ds(sub * NSTR, NSTR)], zbuf.at[pl.ds(0, NSTR)])
    pltpu.sync_copy(zbuf.at[pl.ds(0, NSTR)],
                    degout_ref.at[pl.ds(core * NPAD + sub * NSTR, NSTR)])
    pltpu.sync_copy(cacc.at[pl.ds(sub * GSTR, GSTR)], zbuf.at[pl.ds(0, GSTR)])
    pltpu.sync_copy(zbuf.at[pl.ds(0, GSTR)],
                    cntout_ref.at[pl.ds(core * GPAD + sub * GSTR, GSTR)])


@functools.cache
def _build_s1():
    return pl.kernel(
        _s1_body,
        out_type=(jax.ShapeDtypeStruct((NC * NPAD,), jnp.float32),
                  jax.ShapeDtypeStruct((NC * GPAD,), jnp.float32)),
        mesh=_mesh(),
        compiler_params=pltpu.CompilerParams(use_tc_tiling_on_sc=False),
        scratch_types=[
            pltpu.VMEM((S1_CH // 128, 128), jnp.int32),
            pltpu.VMEM((S1_CH // 128, 128), jnp.float32),
            pltpu.VMEM((PPAD // NW // 128, 128), jnp.int32),
            pltpu.VMEM((NSTR,), jnp.float32),
            pltpu.VMEM((128,), jnp.float32),
            pltpu.VMEM_SHARED((NPAD,), jnp.float32),
            pltpu.VMEM_SHARED((GPAD,), jnp.float32),
        ])


def _s1(*args):
    return _build_s1()(*args)


def _s2_body(table_ref, src_ref, dst_ref, w_ref, out_ref,
             srcv, dstv, wv, rows, sem, acc):
    core = lax.axis_index("c")
    sub = lax.axis_index("s")
    wid = sub * NC + core

    def zrow(i, carry):
        rows[i, pl.ds(0, 16)] = jnp.zeros((16,), jnp.float32)
        rows[i, pl.ds(16, 16)] = jnp.zeros((16,), jnp.float32)
        return carry
    lax.fori_loop(0, S2_CH, zrow, 0)
    for k in range(7):
        pltpu.sync_copy(rows.at[pl.ds(0, 448)],
                        acc.at[pl.ds(sub * NSTR + k * 448, 448)])
    plsc.subcore_barrier()

    def chunk(ci, carry):
        row0 = wid * (EPT // 128) + ci * (S2_CH // 128)
        pltpu.sync_copy(src_ref.at[pl.ds(row0, S2_CH // 128)], srcv)
        pltpu.sync_copy(dst_ref.at[pl.ds(row0, S2_CH // 128)], dstv)
        pltpu.sync_copy(w_ref.at[pl.ds(wid * EPT + ci * S2_CH, S2_CH)], wv)
        descs = [pltpu.async_copy(table_ref.at[srcv.at[j]],
                                  rows.at[pl.ds(j * 128, 128)], sem)
                 for j in range(S2_CH // 128)]
        for d in descs:
            d.wait()

        def mul(i, c2):
            w16 = wv[pl.ds(i * 16, 16)]
            for l in range(16):
                r = i * 16 + l
                ws = w16[l]
                rows[r, pl.ds(0, 16)] = rows[r, pl.ds(0, 16)] * ws
                rows[r, pl.ds(16, 16)] = rows[r, pl.ds(16, 16)] * ws
            return c2
        lax.fori_loop(0, S2_CH // 16, mul, 0)
        for j in range(S2_CH // 128):
            pltpu.sync_copy(rows.at[pl.ds(j * 128, 128)],
                            acc.at[dstv.at[j]], add=True)
        return carry
    lax.fori_loop(0, EPT // S2_CH, chunk, 0)
    plsc.subcore_barrier()

    for k in range(7):
        pltpu.sync_copy(acc.at[pl.ds(sub * NSTR + k * 448, 448)],
                        rows.at[pl.ds(0, 448)])
        pltpu.sync_copy(rows.at[pl.ds(0, 448)],
                        out_ref.at[core, pl.ds(sub * NSTR + k * 448, 448)])


@functools.cache
def _build_s2():
    return pl.kernel(
        _s2_body,
        out_type=jax.ShapeDtypeStruct((NC, NPAD, HID), jnp.float32),
        mesh=_mesh(),
        compiler_params=pltpu.CompilerParams(use_tc_tiling_on_sc=False),
        scratch_types=[
            pltpu.VMEM((S2_CH // 128, 128), jnp.int32),
            pltpu.VMEM((S2_CH // 128, 128), jnp.int32),
            pltpu.VMEM((S2_CH,), jnp.float32),
            pltpu.VMEM((S2_CH, HID), jnp.float32),
            pltpu.SemaphoreType.DMA,
            pltpu.VMEM_SHARED((NPAD, HID), jnp.float32),
        ])


def _s2(*args):
    return _build_s2()(*args)


_PPT = PPAD // NW


def _s3_body(x2_ref, batch_ref, out_ref, batchv, rows, acc):
    core = lax.axis_index("c")
    sub = lax.axis_index("s")
    wid = sub * NC + core

    def zrow(i, carry):
        rows[i, pl.ds(0, 16)] = jnp.zeros((16,), jnp.float32)
        rows[i, pl.ds(16, 16)] = jnp.zeros((16,), jnp.float32)
        return carry
    lax.fori_loop(0, GSTR, zrow, 0)
    pltpu.sync_copy(rows.at[pl.ds(0, GSTR)], acc.at[pl.ds(sub * GSTR, GSTR)])
    plsc.subcore_barrier()

    pltpu.sync_copy(x2_ref.at[pl.ds(wid * _PPT, _PPT)], rows)
    for j in range(_PPT // 128):
        pltpu.sync_copy(batch_ref.at[pl.ds(wid * _PPT + j * 128, 128)],
                        batchv.at[j])
    for j in range(_PPT // 128):
        pltpu.sync_copy(rows.at[pl.ds(j * 128, 128)],
                        acc.at[batchv.at[j]], add=True)
    plsc.subcore_barrier()

    pltpu.sync_copy(acc.at[pl.ds(sub * GSTR, GSTR)], rows.at[pl.ds(0, GSTR)])
    pltpu.sync_copy(rows.at[pl.ds(0, GSTR)],
                    out_ref.at[core, pl.ds(sub * GSTR, GSTR)])


@functools.cache
def _build_s3():
    return pl.kernel(
        _s3_body,
        out_type=jax.ShapeDtypeStruct((NC, GPAD, OUT), jnp.float32),
        mesh=_mesh(),
        compiler_params=pltpu.CompilerParams(use_tc_tiling_on_sc=False),
        scratch_types=[
            pltpu.VMEM((_PPT // 128, 128), jnp.int32),
            pltpu.VMEM((_PPT, OUT), jnp.float32),
            pltpu.VMEM_SHARED((GPAD, OUT), jnp.float32),
        ])


def _s3(*args):
    return _build_s3()(*args)


def kernel(x, edge_index, edge_attr, batch, Wih_s, Whh_s, bih_s, bhh_s,
           Watt_s, Wfc_s, bfc_s, Wih_n, Whh_n, bih_n, bhh_n, Watt_n, Wfc_n,
           bfc_n, Wg1, bg1, Wg2, bg2, Wls, bls):
    f32 = jnp.float32
    x = x.astype(f32)

    xT_node = jnp.concatenate(
        [x.T, jnp.zeros((LAG, NPAD - N), f32)], 1).reshape(LAG, 1, NPAD)
    xsT = x.reshape(NG, 3, LAG).transpose(2, 1, 0)
    xsT = jnp.concatenate([xsT, jnp.zeros((LAG, 3, GPAD - NG), f32)], 2)
    src = edge_index[0].astype(jnp.int32)
    dst = edge_index[1].astype(jnp.int32)
    epad = E_PAD - E
    src2 = jnp.concatenate([src, jnp.zeros((epad,), jnp.int32)]).reshape(-1, 128)
    dst2 = jnp.concatenate([dst, jnp.zeros((epad,), jnp.int32)]).reshape(-1, 128)
    w_p = jnp.concatenate([edge_attr.astype(f32), jnp.zeros((epad,), f32)])
    batch1 = jnp.concatenate(
        [batch.astype(jnp.int32), jnp.full((PPAD - N,), NG, jnp.int32)])

    b_n = (bih_n + bhh_n).reshape(4 * HID, 1)
    b_s = (bih_s + bhh_s).reshape(4 * HID, 1)

    emb = _attlstm(xT_node, Wih_n, Whh_n, b_n,
                   Watt_n, Wfc_n, bfc_n.reshape(OUT, 1))
    x1s = _attlstm(xsT, Wih_s, Whh_s, b_s,
                   Watt_s, Wfc_s, bfc_s.reshape(OUT, 1))

    degp, cntp = _s1(dst2, w_p.reshape(-1, 128), batch1)
    degp = degp.reshape(NC, NPAD)
    cntp = cntp.reshape(NC, GPAD)

    dis, h1s = _g0(degp.T, emb, Wg1)
    agg1p = _s2(h1s, src2, dst2, w_p)
    h2s = _g1(agg1p, dis, h1s, bg1.reshape(1, HID), Wg2)

    agg2p = _s2(h2s, src2, dst2, w_p)
    x2 = _g2(agg2p, dis, h2s, bg2.reshape(1, OUT))

    x2_pool = jnp.concatenate([x2, jnp.zeros((PPAD - NPAD, OUT), f32)], 0)
    ssump = _s3(x2_pool, batch1)
    pred = _g3(ssump, cntp.T, x1s, Wls[:OUT].astype(f32),
               Wls[OUT:].astype(f32), bls.reshape(1, 1))
    return pred[:NG, 0]


# speedup vs baseline: 21.3730x; 1.1928x over previous
import functools

import jax
import jax.numpy as jnp
from jax import lax
from jax.experimental import pallas as pl
from jax.experimental.pallas import tpu as pltpu
from jax.experimental.pallas import tpu_sc as plsc

N = 50001
E = 1600032
NG = 16667
LAG = 30
HID = 32
OUT = 32

NC = 2
NS = 16
NW = NC * NS

NPAD = 50176
GPAD = 16896
PPAD = 53248
E_PAD = 1638400
EPT = E_PAD // NW

R_LSTM = 512
S1_CH = 1024
S2_CH = 256
SZB = 224
NSTR = NPAD // NS
GSTR = GPAD // NS

@functools.cache
def _mesh():
    return plsc.VectorSubcoreMesh(
        core_axis_name="c", subcore_axis_name="s",
        num_cores=NC, num_subcores=NS)


def _attlstm_body(x_ref, wih_ref, whh_ref, b_ref, watt_ref, wfc_ref,
                  bfc_ref, out_ref, hs_ref):
    R = x_ref.shape[2]
    D = x_ref.shape[1]
    whh = whh_ref[...]
    b = b_ref[...]
    hT = jnp.zeros((HID, R), jnp.float32)
    cT = jnp.zeros((HID, R), jnp.float32)
    scores = []
    for t in range(LAG):
        xtT = x_ref[t]
        if D == 1:
            gx = wih_ref[...] * xtT
        else:
            gx = jnp.dot(wih_ref[...], xtT,
                         preferred_element_type=jnp.float32)
        g = gx + jnp.dot(whh, hT, preferred_element_type=jnp.float32) + b
        i = jax.nn.sigmoid(g[0:HID])
        f = jax.nn.sigmoid(g[HID:2 * HID])
        gg = jnp.tanh(g[2 * HID:3 * HID])
        o = jax.nn.sigmoid(g[3 * HID:4 * HID])
        cT = f * cT + i * gg
        hT = o * jnp.tanh(cT)
        hs_ref[t] = hT
        scores.append(jnp.dot(watt_ref[t:t + 1, :], hT,
                              preferred_element_type=jnp.float32))
    s = jnp.concatenate(scores, axis=0)
    m = jnp.max(s, axis=0, keepdims=True)
    e = jnp.exp(s - m)
    a = e / jnp.sum(e, axis=0, keepdims=True)
    attT = jnp.zeros((HID, R), jnp.float32)
    for t in range(LAG):
        attT = attT + a[t:t + 1, :] * hs_ref[t]
    outT = (jnp.dot(wfc_ref[...], attT, preferred_element_type=jnp.float32)
            + bfc_ref[...])
    out_ref[...] = jnp.maximum(outT, 0.0)


def _attlstm(xpT, wih, whh, b2, watt, wfc, bfc2):
    _, D, B = xpT.shape
    grid = B // R_LSTM
    outT = pl.pallas_call(
        _attlstm_body,
        grid=(grid,),
        in_specs=[
            pl.BlockSpec((LAG, D, R_LSTM), lambda i: (0, 0, i)),
            pl.BlockSpec((4 * HID, D), lambda i: (0, 0)),
            pl.BlockSpec((4 * HID, HID), lambda i: (0, 0)),
            pl.BlockSpec((4 * HID, 1), lambda i: (0, 0)),
            pl.BlockSpec((LAG, HID), lambda i: (0, 0)),
            pl.BlockSpec((OUT, HID), lambda i: (0, 0)),
            pl.BlockSpec((OUT, 1), lambda i: (0, 0)),
        ],
        out_specs=pl.BlockSpec((OUT, R_LSTM), lambda i: (0, i)),
        out_shape=jax.ShapeDtypeStruct((OUT, B), jnp.float32),
        scratch_shapes=[pltpu.VMEM((LAG, HID, R_LSTM), jnp.float32)],
    )(xpT, wih, whh, b2, watt, wfc, bfc2)
    return outT.T


_RG = 3584


def _g0_body(degp_ref, emb_ref, wg1_ref, dis_ref, h1s_ref):
    deg = degp_ref[:, 0:1] + degp_ref[:, 1:2] + 1.0
    dis = lax.rsqrt(deg)
    h1 = jnp.dot(emb_ref[...], wg1_ref[...], preferred_element_type=jnp.float32)
    dis_ref[...] = dis
    h1s_ref[...] = dis * h1


def _g0(degpT, emb, wg1):
    grid = NPAD // _RG
    return pl.pallas_call(
        _g0_body,
        grid=(grid,),
        in_specs=[
            pl.BlockSpec((_RG, NC), lambda i: (i, 0)),
            pl.BlockSpec((_RG, HID), lambda i: (i, 0)),
            pl.BlockSpec((HID, HID), lambda i: (0, 0)),
        ],
        out_specs=[
            pl.BlockSpec((_RG, 1), lambda i: (i, 0)),
            pl.BlockSpec((_RG, HID), lambda i: (i, 0)),
        ],
        out_shape=[
            jax.ShapeDtypeStruct((NPAD, 1), jnp.float32),
            jax.ShapeDtypeStruct((NPAD, HID), jnp.float32),
        ],
    )(degpT, emb, wg1)


def _g1_body(aggp_ref, dis_ref, h1s_ref, bg1_ref, wg2_ref, h2s_ref):
    dis = dis_ref[...]
    x2a = dis * (aggp_ref[0] + aggp_ref[1] + h1s_ref[...]) + bg1_ref[...]
    h2 = jnp.dot(x2a, wg2_ref[...], preferred_element_type=jnp.float32)
    h2s_ref[...] = dis * h2


def _g1(aggp, dis, h1s, bg1, wg2):
    grid = NPAD // _RG
    return pl.pallas_call(
        _g1_body,
        grid=(grid,),
        in_specs=[
            pl.BlockSpec((NC, _RG, HID), lambda i: (0, i, 0)),
            pl.BlockSpec((_RG, 1), lambda i: (i, 0)),
            pl.BlockSpec((_RG, HID), lambda i: (i, 0)),
            pl.BlockSpec((1, HID), lambda i: (0, 0)),
            pl.BlockSpec((HID, OUT), lambda i: (0, 0)),
        ],
        out_specs=pl.BlockSpec((_RG, OUT), lambda i: (i, 0)),
        out_shape=jax.ShapeDtypeStruct((NPAD, OUT), jnp.float32),
    )(aggp, dis, h1s, bg1, wg2)


def _g2_body(aggp_ref, dis_ref, h2s_ref, bg2_ref, x2_ref):
    dis = dis_ref[...]
    x2_ref[...] = dis * (aggp_ref[0] + aggp_ref[1] + h2s_ref[...]) + bg2_ref[...]


def _g2(aggp, dis, h2s, bg2):
    grid = NPAD // _RG
    return pl.pallas_call(
        _g2_body,
        grid=(grid,),
        in_specs=[
            pl.BlockSpec((NC, _RG, OUT), lambda i: (0, i, 0)),
            pl.BlockSpec((_RG, 1), lambda i: (i, 0)),
            pl.BlockSpec((_RG, OUT), lambda i: (i, 0)),
            pl.BlockSpec((1, OUT), lambda i: (0, 0)),
        ],
        out_specs=pl.BlockSpec((_RG, OUT), lambda i: (i, 0)),
        out_shape=jax.ShapeDtypeStruct((NPAD, OUT), jnp.float32),
    )(aggp, dis, h2s, bg2)


_RG3 = 2112


def _g3_body(ssump_ref, cntp_ref, x1s_ref, wa_ref, wb_ref, bls_ref, out_ref):
    cnt = jnp.maximum(cntp_ref[:, 0:1] + cntp_ref[:, 1:2], 1.0)
    x2n = (ssump_ref[0] + ssump_ref[1]) / cnt
    pred = (jnp.dot(x1s_ref[...], wa_ref[...], preferred_element_type=jnp.float32)
            + jnp.dot(x2n, wb_ref[...], preferred_element_type=jnp.float32)
            + bls_ref[...])
    out_ref[...] = jnp.maximum(pred, 0.0)


def _g3(ssump, cntpT, x1s, wa, wb, bls2):
    grid = GPAD // _RG3
    return pl.pallas_call(
        _g3_body,
        grid=(grid,),
        in_specs=[
            pl.BlockSpec((NC, _RG3, OUT), lambda i: (0, i, 0)),
            pl.BlockSpec((_RG3, NC), lambda i: (i, 0)),
            pl.BlockSpec((_RG3, OUT), lambda i: (i, 0)),
            pl.BlockSpec((OUT, 1), lambda i: (0, 0)),
            pl.BlockSpec((OUT, 1), lambda i: (0, 0)),
            pl.BlockSpec((1, 1), lambda i: (0, 0)),
        ],
        out_specs=pl.BlockSpec((_RG3, 1), lambda i: (i, 0)),
        out_shape=jax.ShapeDtypeStruct((GPAD, 1), jnp.float32),
    )(ssump, cntpT, x1s, wa, wb, bls2)


def _memset(ref, n, val):
    def step(i, carry):
        ref[pl.ds(i * 16, 16)] = jnp.full((16,), val, jnp.float32)
        return carry
    lax.fori_loop(0, n // 16, step, 0)


def _s1_body(dst_ref, w_ref, batch_ref, degout_ref, cntout_ref,
             dstv, wv, batchv, zbuf, onesv, dacc, cacc):
    core = lax.axis_index("c")
    sub = lax.axis_index("s")
    wid = sub * NC + core
    _memset(zbuf, NSTR, 0.0)
    _memset(onesv, 128, 1.0)
    pltpu.sync_copy(zbuf.at[pl.ds(0, NSTR)], dacc.at[pl.ds(sub * NSTR, NSTR)])
    pltpu.sync_copy(zbuf.at[pl.ds(0, GSTR)], cacc.at[pl.ds(sub * GSTR, GSTR)])
    plsc.subcore_barrier()

    def echunk(ci, carry):
        row0 = wid * (EPT // 128) + ci * (S1_CH // 128)
        pltpu.sync_copy(dst_ref.at[pl.ds(row0, S1_CH // 128)], dstv)
        pltpu.sync_copy(w_ref.at[pl.ds(row0, S1_CH // 128)], wv)
        for j in range(S1_CH // 128):
            pltpu.sync_copy(wv.at[j], dacc.at[dstv.at[j]], add=True)
        return carry
    lax.fori_loop(0, EPT // S1_CH, echunk, 0)

    for j in range(PPAD // NW // 128):
        pltpu.sync_copy(batch_ref.at[pl.ds(wid * (PPAD // NW) + j * 128, 128)],
                        batchv.at[j])
    for j in range(PPAD // NW // 128):
        pltpu.sync_copy(onesv, cacc.at[batchv.at[j]], add=True)
    plsc.subcore_barrier()

    pltpu.sync_copy(dacc.at[pl.ds(sub * NSTR, NSTR)], zbuf.at[pl.ds(0, NSTR)])
    pltpu.sync_copy(zbuf.at[pl.ds(0, NSTR)],
                    degout_ref.at[pl.ds(core * NPAD + sub * NSTR, NSTR)])
    pltpu.sync_copy(cacc.at[pl.ds(sub * GSTR, GSTR)], zbuf.at[pl.ds(0, GSTR)])
    pltpu.sync_copy(zbuf.at[pl.ds(0, GSTR)],
                    cntout_ref.at[pl.ds(core * GPAD + sub * GSTR, GSTR)])


@functools.cache
def _build_s1():
    return pl.kernel(
        _s1_body,
        out_type=(jax.ShapeDtypeStruct((NC * NPAD,), jnp.float32),
                  jax.ShapeDtypeStruct((NC * GPAD,), jnp.float32)),
        mesh=_mesh(),
        compiler_params=pltpu.CompilerParams(use_tc_tiling_on_sc=False),
        scratch_types=[
            pltpu.VMEM((S1_CH // 128, 128), jnp.int32),
            pltpu.VMEM((S1_CH // 128, 128), jnp.float32),
            pltpu.VMEM((PPAD // NW // 128, 128), jnp.int32),
            pltpu.VMEM((NSTR,), jnp.float32),
            pltpu.VMEM((128,), jnp.float32),
            pltpu.VMEM_SHARED((NPAD,), jnp.float32),
            pltpu.VMEM_SHARED((GPAD,), jnp.float32),
        ])


def _s1(*args):
    return _build_s1()(*args)


_NJ = S2_CH // 128
_NCH = EPT // S2_CH


def _s2_body(table_ref, src_ref, dst_ref, w_ref, out_ref,
             srcv, dstv, wv, rows, semr0, semr1, semi0, semi1, acc):
    core = lax.axis_index("c")
    sub = lax.axis_index("s")
    wid = sub * NC + core
    semr = (semr0, semr1)
    semi = (semi0, semi1)

    def zrow(i, carry):
        rows[0, i, pl.ds(0, 16)] = jnp.zeros((16,), jnp.float32)
        rows[0, i, pl.ds(16, 16)] = jnp.zeros((16,), jnp.float32)
        return carry
    lax.fori_loop(0, S2_CH, zrow, 0)
    for k in range(NSTR // SZB):
        pltpu.sync_copy(rows.at[0, pl.ds(0, SZB)],
                        acc.at[pl.ds(sub * NSTR + k * SZB, SZB)])
    plsc.subcore_barrier()

    def idx_descs(c, b):
        base = wid * EPT + c * S2_CH
        ds_ = []
        for j in range(_NJ):
            ds_.append(pltpu.make_async_copy(
                src_ref.at[pl.ds(base + j * 128, 128)],
                srcv.at[_NJ * b + j], semi[b]))
            ds_.append(pltpu.make_async_copy(
                dst_ref.at[pl.ds(base + j * 128, 128)],
                dstv.at[_NJ * b + j], semi[b]))
        ds_.append(pltpu.make_async_copy(
            w_ref.at[pl.ds(base, S2_CH)], wv.at[b], semi[b]))
        return ds_

    def gather_descs(c, b):
        return [pltpu.make_async_copy(table_ref.at[srcv.at[_NJ * b + j]],
                                      rows.at[b, pl.ds(j * 128, 128)],
                                      semr[b])
                for j in range(_NJ)]

    def fire(descs):
        for d_ in descs:
            d_.start()

    def drain(descs):
        for d_ in descs:
            d_.wait()

    def process(c, b, fire_gnext, fire_inext):
        if fire_gnext:
            drain(idx_descs(c + 1, 1 - b))
            fire(gather_descs(c + 1, 1 - b))
        drain(gather_descs(c, b))

        def mul(i, c2):
            w16 = wv[b, pl.ds(i * 16, 16)]
            for l in range(16):
                r = i * 16 + l
                rows[b, r, pl.ds(0, 16)] = rows[b, r, pl.ds(0, 16)] * w16[l]
                rows[b, r, pl.ds(16, 16)] = rows[b, r, pl.ds(16, 16)] * w16[l]
            return c2
        lax.fori_loop(0, S2_CH // 16, mul, 0)
        for j in range(_NJ):
            pltpu.sync_copy(rows.at[b, pl.ds(j * 128, 128)],
                            acc.at[dstv.at[_NJ * b + j]], add=True)
        if fire_inext:
            fire(idx_descs(c + 2, b))

    fire(idx_descs(0, 0))
    drain(idx_descs(0, 0))
    fire(gather_descs(0, 0))
    fire(idx_descs(1, 1))

    def body(cj, carry):
        c = cj * 2
        process(c, 0, True, True)
        process(c + 1, 1, True, True)
        return carry
    lax.fori_loop(0, _NCH // 2 - 1, body, 0)
    process(_NCH - 2, 0, True, False)
    process(_NCH - 1, 1, False, False)
    plsc.subcore_barrier()

    for k in range(NSTR // SZB):
        pltpu.sync_copy(acc.at[pl.ds(sub * NSTR + k * SZB, SZB)],
                        rows.at[0, pl.ds(0, SZB)])
        pltpu.sync_copy(rows.at[0, pl.ds(0, SZB)],
                        out_ref.at[core, pl.ds(sub * NSTR + k * SZB, SZB)])


@functools.cache
def _build_s2():
    return pl.kernel(
        _s2_body,
        out_type=jax.ShapeDtypeStruct((NC, NPAD, HID), jnp.float32),
        mesh=_mesh(),
        compiler_params=pltpu.CompilerParams(use_tc_tiling_on_sc=False),
        scratch_types=[
            pltpu.VMEM((2 * _NJ, 128), jnp.int32),
            pltpu.VMEM((2 * _NJ, 128), jnp.int32),
            pltpu.VMEM((2, S2_CH), jnp.float32),
            pltpu.VMEM((2, S2_CH, HID), jnp.float32),
            pltpu.SemaphoreType.DMA,
            pltpu.SemaphoreType.DMA,
            pltpu.SemaphoreType.DMA,
            pltpu.SemaphoreType.DMA,
            pltpu.VMEM_SHARED((NPAD, HID), jnp.float32),
        ])


def _s2(*args):
    return _build_s2()(*args)


_PPT = PPAD // NW


def _s3_body(x2_ref, batch_ref, out_ref, batchv, rows, acc):
    core = lax.axis_index("c")
    sub = lax.axis_index("s")
    wid = sub * NC + core

    def zrow(i, carry):
        rows[i, pl.ds(0, 16)] = jnp.zeros((16,), jnp.float32)
        rows[i, pl.ds(16, 16)] = jnp.zeros((16,), jnp.float32)
        return carry
    lax.fori_loop(0, GSTR, zrow, 0)
    pltpu.sync_copy(rows.at[pl.ds(0, GSTR)], acc.at[pl.ds(sub * GSTR, GSTR)])
    plsc.subcore_barrier()

    pltpu.sync_copy(x2_ref.at[pl.ds(wid * _PPT, _PPT)], rows)
    for j in range(_PPT // 128):
        pltpu.sync_copy(batch_ref.at[pl.ds(wid * _PPT + j * 128, 128)],
                        batchv.at[j])
    for j in range(_PPT // 128):
        pltpu.sync_copy(rows.at[pl.ds(j * 128, 128)],
                        acc.at[batchv.at[j]], add=True)
    plsc.subcore_barrier()

    pltpu.sync_copy(acc.at[pl.ds(sub * GSTR, GSTR)], rows.at[pl.ds(0, GSTR)])
    pltpu.sync_copy(rows.at[pl.ds(0, GSTR)],
                    out_ref.at[core, pl.ds(sub * GSTR, GSTR)])


@functools.cache
def _build_s3():
    return pl.kernel(
        _s3_body,
        out_type=jax.ShapeDtypeStruct((NC, GPAD, OUT), jnp.float32),
        mesh=_mesh(),
        compiler_params=pltpu.CompilerParams(use_tc_tiling_on_sc=False),
        scratch_types=[
            pltpu.VMEM((_PPT // 128, 128), jnp.int32),
            pltpu.VMEM((_PPT, OUT), jnp.float32),
            pltpu.VMEM_SHARED((GPAD, OUT), jnp.float32),
        ])


def _s3(*args):
    return _build_s3()(*args)


def kernel(x, edge_index, edge_attr, batch, Wih_s, Whh_s, bih_s, bhh_s,
           Watt_s, Wfc_s, bfc_s, Wih_n, Whh_n, bih_n, bhh_n, Watt_n, Wfc_n,
           bfc_n, Wg1, bg1, Wg2, bg2, Wls, bls):
    f32 = jnp.float32
    x = x.astype(f32)

    xT_node = jnp.concatenate(
        [x.T, jnp.zeros((LAG, NPAD - N), f32)], 1).reshape(LAG, 1, NPAD)
    xsT = x.reshape(NG, 3, LAG).transpose(2, 1, 0)
    xsT = jnp.concatenate([xsT, jnp.zeros((LAG, 3, GPAD - NG), f32)], 2)
    src = edge_index[0].astype(jnp.int32)
    dst = edge_index[1].astype(jnp.int32)
    epad = E_PAD - E
    src_p = jnp.concatenate([src, jnp.zeros((epad,), jnp.int32)])
    dst_p = jnp.concatenate([dst, jnp.zeros((epad,), jnp.int32)])
    dst2 = dst_p.reshape(-1, 128)
    w_p = jnp.concatenate([edge_attr.astype(f32), jnp.zeros((epad,), f32)])
    batch1 = jnp.concatenate(
        [batch.astype(jnp.int32), jnp.full((PPAD - N,), NG, jnp.int32)])

    b_n = (bih_n + bhh_n).reshape(4 * HID, 1)
    b_s = (bih_s + bhh_s).reshape(4 * HID, 1)

    emb = _attlstm(xT_node, Wih_n, Whh_n, b_n,
                   Watt_n, Wfc_n, bfc_n.reshape(OUT, 1))

    degp, cntp = _s1(dst2, w_p.reshape(-1, 128), batch1)
    degp = degp.reshape(NC, NPAD)
    cntp = cntp.reshape(NC, GPAD)

    dis, h1s = _g0(degp.T, emb, Wg1)
    agg1p = _s2(h1s, src_p, dst_p, w_p)
    h2s = _g1(agg1p, dis, h1s, bg1.reshape(1, HID), Wg2)

    xsT_b, _ = lax.optimization_barrier((xsT, agg1p))
    x1s = _attlstm(xsT_b, Wih_s, Whh_s, b_s,
                   Watt_s, Wfc_s, bfc_s.reshape(OUT, 1))

    agg2p = _s2(h2s, src_p, dst_p, w_p)
    x2 = _g2(agg2p, dis, h2s, bg2.reshape(1, OUT))

    x2_pool = jnp.concatenate([x2, jnp.zeros((PPAD - NPAD, OUT), f32)], 0)
    ssump = _s3(x2_pool, batch1)
    pred = _g3(ssump, cntp.T, x1s, Wls[:OUT].astype(f32),
               Wls[OUT:].astype(f32), bls.reshape(1, 1))
    return pred[:NG, 0]


# speedup vs baseline: 22.6670x; 1.0605x over previous
import functools

import jax
import jax.numpy as jnp
from jax import lax
from jax.experimental import pallas as pl
from jax.experimental.pallas import tpu as pltpu
from jax.experimental.pallas import tpu_sc as plsc

N = 50001
E = 1600032
NG = 16667
LAG = 30
HID = 32
OUT = 32

NC = 2
NS = 16
NW = NC * NS

NPAD = 50176
GPAD = 16896
PPAD = 53248
E_PAD = 1638400
EPT = E_PAD // NW

R_LSTM = 512
S1_CH = 1024
S2_CH = 256
SZB = 224
NSTR = NPAD // NS
GSTR = GPAD // NS

@functools.cache
def _mesh():
    return plsc.VectorSubcoreMesh(
        core_axis_name="c", subcore_axis_name="s",
        num_cores=NC, num_subcores=NS)


def _attlstm_body(x_ref, wih_ref, whh_ref, b_ref, watt_ref, wfc_ref,
                  bfc_ref, out_ref, hs_ref):
    R = x_ref.shape[2]
    D = x_ref.shape[1]
    whh = whh_ref[...]
    b = b_ref[...]
    hT = jnp.zeros((HID, R), jnp.float32)
    cT = jnp.zeros((HID, R), jnp.float32)
    scores = []
    for t in range(LAG):
        xtT = x_ref[t]
        if D == 1:
            gx = wih_ref[...] * xtT
        else:
            gx = jnp.dot(wih_ref[...], xtT,
                         preferred_element_type=jnp.float32)
        g = gx + jnp.dot(whh, hT, preferred_element_type=jnp.float32) + b
        i = jax.nn.sigmoid(g[0:HID])
        f = jax.nn.sigmoid(g[HID:2 * HID])
        gg = jnp.tanh(g[2 * HID:3 * HID])
        o = jax.nn.sigmoid(g[3 * HID:4 * HID])
        cT = f * cT + i * gg
        hT = o * jnp.tanh(cT)
        hs_ref[t] = hT
        scores.append(jnp.dot(watt_ref[t:t + 1, :], hT,
                              preferred_element_type=jnp.float32))
    s = jnp.concatenate(scores, axis=0)
    m = jnp.max(s, axis=0, keepdims=True)
    e = jnp.exp(s - m)
    a = e / jnp.sum(e, axis=0, keepdims=True)
    attT = jnp.zeros((HID, R), jnp.float32)
    for t in range(LAG):
        attT = attT + a[t:t + 1, :] * hs_ref[t]
    outT = (jnp.dot(wfc_ref[...], attT, preferred_element_type=jnp.float32)
            + bfc_ref[...])
    out_ref[...] = jnp.maximum(outT, 0.0)


def _attlstm(xpT, wih, whh, b2, watt, wfc, bfc2):
    _, D, B = xpT.shape
    grid = B // R_LSTM
    outT = pl.pallas_call(
        _attlstm_body,
        grid=(grid,),
        in_specs=[
            pl.BlockSpec((LAG, D, R_LSTM), lambda i: (0, 0, i)),
            pl.BlockSpec((4 * HID, D), lambda i: (0, 0)),
            pl.BlockSpec((4 * HID, HID), lambda i: (0, 0)),
            pl.BlockSpec((4 * HID, 1), lambda i: (0, 0)),
            pl.BlockSpec((LAG, HID), lambda i: (0, 0)),
            pl.BlockSpec((OUT, HID), lambda i: (0, 0)),
            pl.BlockSpec((OUT, 1), lambda i: (0, 0)),
        ],
        out_specs=pl.BlockSpec((OUT, R_LSTM), lambda i: (0, i)),
        out_shape=jax.ShapeDtypeStruct((OUT, B), jnp.float32),
        scratch_shapes=[pltpu.VMEM((LAG, HID, R_LSTM), jnp.float32)],
    )(xpT, wih, whh, b2, watt, wfc, bfc2)
    return outT.T


_RG = 3584


def _g0_body(degp_ref, emb_ref, wg1_ref, dis_ref, h1s_ref):
    deg = degp_ref[:, 0:1] + degp_ref[:, 1:2] + 1.0
    dis = lax.rsqrt(deg)
    h1 = jnp.dot(emb_ref[...], wg1_ref[...], preferred_element_type=jnp.float32)
    dis_ref[...] = dis
    h1s_ref[...] = dis * h1


def _g0(degpT, emb, wg1):
    grid = NPAD // _RG
    return pl.pallas_call(
        _g0_body,
        grid=(grid,),
        in_specs=[
            pl.BlockSpec((_RG, NC), lambda i: (i, 0)),
            pl.BlockSpec((_RG, HID), lambda i: (i, 0)),
            pl.BlockSpec((HID, HID), lambda i: (0, 0)),
        ],
        out_specs=[
            pl.BlockSpec((_RG, 1), lambda i: (i, 0)),
            pl.BlockSpec((_RG, HID), lambda i: (i, 0)),
        ],
        out_shape=[
            jax.ShapeDtypeStruct((NPAD, 1), jnp.float32),
            jax.ShapeDtypeStruct((NPAD, HID), jnp.float32),
        ],
    )(degpT, emb, wg1)


def _g1_body(aggp_ref, dis_ref, h1s_ref, bg1_ref, wg2_ref, h2s_ref):
    dis = dis_ref[...]
    x2a = dis * (aggp_ref[0] + aggp_ref[1] + h1s_ref[...]) + bg1_ref[...]
    h2 = jnp.dot(x2a, wg2_ref[...], preferred_element_type=jnp.float32)
    h2s_ref[...] = dis * h2


def _g1(aggp, dis, h1s, bg1, wg2):
    grid = NPAD // _RG
    return pl.pallas_call(
        _g1_body,
        grid=(grid,),
        in_specs=[
            pl.BlockSpec((NC, _RG, HID), lambda i: (0, i, 0)),
            pl.BlockSpec((_RG, 1), lambda i: (i, 0)),
            pl.BlockSpec((_RG, HID), lambda i: (i, 0)),
            pl.BlockSpec((1, HID), lambda i: (0, 0)),
            pl.BlockSpec((HID, OUT), lambda i: (0, 0)),
        ],
        out_specs=pl.BlockSpec((_RG, OUT), lambda i: (i, 0)),
        out_shape=jax.ShapeDtypeStruct((NPAD, OUT), jnp.float32),
    )(aggp, dis, h1s, bg1, wg2)


def _g2_body(aggp_ref, dis_ref, h2s_ref, bg2_ref, x2_ref):
    dis = dis_ref[...]
    x2_ref[...] = dis * (aggp_ref[0] + aggp_ref[1] + h2s_ref[...]) + bg2_ref[...]


def _g2(aggp, dis, h2s, bg2):
    grid = NPAD // _RG
    return pl.pallas_call(
        _g2_body,
        grid=(grid,),
        in_specs=[
            pl.BlockSpec((NC, _RG, OUT), lambda i: (0, i, 0)),
            pl.BlockSpec((_RG, 1), lambda i: (i, 0)),
            pl.BlockSpec((_RG, OUT), lambda i: (i, 0)),
            pl.BlockSpec((1, OUT), lambda i: (0, 0)),
        ],
        out_specs=pl.BlockSpec((_RG, OUT), lambda i: (i, 0)),
        out_shape=jax.ShapeDtypeStruct((NPAD, OUT), jnp.float32),
    )(aggp, dis, h2s, bg2)


_RG3 = 2112


def _g3_body(ssump_ref, cntp_ref, x1s_ref, wa_ref, wb_ref, bls_ref, out_ref):
    cnt = jnp.maximum(cntp_ref[:, 0:1] + cntp_ref[:, 1:2], 1.0)
    x2n = (ssump_ref[0] + ssump_ref[1]) / cnt
    pred = (jnp.dot(x1s_ref[...], wa_ref[...], preferred_element_type=jnp.float32)
            + jnp.dot(x2n, wb_ref[...], preferred_element_type=jnp.float32)
            + bls_ref[...])
    out_ref[...] = jnp.maximum(pred, 0.0)


def _g3(ssump, cntpT, x1s, wa, wb, bls2):
    grid = GPAD // _RG3
    return pl.pallas_call(
        _g3_body,
        grid=(grid,),
        in_specs=[
            pl.BlockSpec((NC, _RG3, OUT), lambda i: (0, i, 0)),
            pl.BlockSpec((_RG3, NC), lambda i: (i, 0)),
            pl.BlockSpec((_RG3, OUT), lambda i: (i, 0)),
            pl.BlockSpec((OUT, 1), lambda i: (0, 0)),
            pl.BlockSpec((OUT, 1), lambda i: (0, 0)),
            pl.BlockSpec((1, 1), lambda i: (0, 0)),
        ],
        out_specs=pl.BlockSpec((_RG3, 1), lambda i: (i, 0)),
        out_shape=jax.ShapeDtypeStruct((GPAD, 1), jnp.float32),
    )(ssump, cntpT, x1s, wa, wb, bls2)


_PCH = 6400


def _s0_body(src_ref, dst_ref, w_ref, srco_ref, dsto_ref, wo_ref, bi, bf):
    core = lax.axis_index("c")
    sub = lax.axis_index("s")
    wid = sub * NC + core

    def chunk(ci, carry):
        base = wid * EPT + ci * _PCH
        for a, o, b in ((src_ref, srco_ref, bi), (dst_ref, dsto_ref, bi),
                        (w_ref, wo_ref, bf)):
            pltpu.sync_copy(a.at[pl.ds(base, _PCH)], b)
            pltpu.sync_copy(b, o.at[pl.ds(base, _PCH)])
        return carry
    lax.fori_loop(0, EPT // _PCH, chunk, 0)


@functools.cache
def _build_s0():
    return pl.kernel(
        _s0_body,
        out_type=(jax.ShapeDtypeStruct((E_PAD,), jnp.int32),
                  jax.ShapeDtypeStruct((E_PAD,), jnp.int32),
                  jax.ShapeDtypeStruct((E_PAD,), jnp.float32)),
        mesh=_mesh(),
        compiler_params=pltpu.CompilerParams(use_tc_tiling_on_sc=False),
        scratch_types=[pltpu.VMEM((_PCH,), jnp.int32),
                       pltpu.VMEM((_PCH,), jnp.float32)])


def _s0(*args):
    return _build_s0()(*args)


def _memset(ref, n, val):
    def step(i, carry):
        ref[pl.ds(i * 16, 16)] = jnp.full((16,), val, jnp.float32)
        return carry
    lax.fori_loop(0, n // 16, step, 0)


def _s1_body(dst_ref, w_ref, batch_ref, degout_ref, cntout_ref,
             dstv, wv, batchv, zbuf, onesv, dacc, cacc):
    core = lax.axis_index("c")
    sub = lax.axis_index("s")
    wid = sub * NC + core
    _memset(zbuf, NSTR, 0.0)
    _memset(onesv, 128, 1.0)
    pltpu.sync_copy(zbuf.at[pl.ds(0, NSTR)], dacc.at[pl.ds(sub * NSTR, NSTR)])
    pltpu.sync_copy(zbuf.at[pl.ds(0, GSTR)], cacc.at[pl.ds(sub * GSTR, GSTR)])
    plsc.subcore_barrier()

    def echunk(ci, carry):
        row0 = wid * (EPT // 128) + ci * (S1_CH // 128)
        pltpu.sync_copy(dst_ref.at[pl.ds(row0, S1_CH // 128)], dstv)
        pltpu.sync_copy(w_ref.at[pl.ds(row0, S1_CH // 128)], wv)
        for j in range(S1_CH // 128):
            pltpu.sync_copy(wv.at[j], dacc.at[dstv.at[j]], add=True)
        return carry
    lax.fori_loop(0, EPT // S1_CH, echunk, 0)

    for j in range(PPAD // NW // 128):
        pltpu.sync_copy(batch_ref.at[pl.ds(wid * (PPAD // NW) + j * 128, 128)],
                        batchv.at[j])
    for j in range(PPAD // NW // 128):
        pltpu.sync_copy(onesv, cacc.at[batchv.at[j]], add=True)
    plsc.subcore_barrier()

    pltpu.sync_copy(dacc.at[pl.ds(sub * NSTR, NSTR)], zbuf.at[pl.ds(0, NSTR)])
    pltpu.sync_copy(zbuf.at[pl.ds(0, NSTR)],
                    degout_ref.at[pl.ds(core * NPAD + sub * NSTR, NSTR)])
    pltpu.sync_copy(cacc.at[pl.ds(sub * GSTR, GSTR)], zbuf.at[pl.ds(0, GSTR)])
    pltpu.sync_copy(zbuf.at[pl.ds(0, GSTR)],
                    cntout_ref.at[pl.ds(core * GPAD + sub * GSTR, GSTR)])


@functools.cache
def _build_s1():
    return pl.kernel(
        _s1_body,
        out_type=(jax.ShapeDtypeStruct((NC * NPAD,), jnp.float32),
                  jax.ShapeDtypeStruct((NC * GPAD,), jnp.float32)),
        mesh=_mesh(),
        compiler_params=pltpu.CompilerParams(use_tc_tiling_on_sc=False),
        scratch_types=[
            pltpu.VMEM((S1_CH // 128, 128), jnp.int32),
            pltpu.VMEM((S1_CH // 128, 128), jnp.float32),
            pltpu.VMEM((PPAD // NW // 128, 128), jnp.int32),
            pltpu.VMEM((NSTR,), jnp.float32),
            pltpu.VMEM((128,), jnp.float32),
            pltpu.VMEM_SHARED((NPAD,), jnp.float32),
            pltpu.VMEM_SHARED((GPAD,), jnp.float32),
        ])


def _s1(*args):
    return _build_s1()(*args)


_NJ = S2_CH // 128
_NCH = EPT // S2_CH


def _s2_body(table_ref, src_ref, dst_ref, w_ref, out_ref,
             srcv, dstv, wv, rows, semr0, semr1, semi0, semi1, acc):
    core = lax.axis_index("c")
    sub = lax.axis_index("s")
    wid = sub * NC + core
    semr = (semr0, semr1)
    semi = (semi0, semi1)

    def zrow(i, carry):
        rows[0, i, pl.ds(0, 16)] = jnp.zeros((16,), jnp.float32)
        rows[0, i, pl.ds(16, 16)] = jnp.zeros((16,), jnp.float32)
        return carry
    lax.fori_loop(0, S2_CH, zrow, 0)
    for k in range(NSTR // SZB):
        pltpu.sync_copy(rows.at[0, pl.ds(0, SZB)],
                        acc.at[pl.ds(sub * NSTR + k * SZB, SZB)])
    plsc.subcore_barrier()

    def idx_descs(c, b):
        base = wid * EPT + c * S2_CH
        ds_ = []
        for j in range(_NJ):
            ds_.append(pltpu.make_async_copy(
                src_ref.at[pl.ds(base + j * 128, 128)],
                srcv.at[_NJ * b + j], semi[b]))
            ds_.append(pltpu.make_async_copy(
                dst_ref.at[pl.ds(base + j * 128, 128)],
                dstv.at[_NJ * b + j], semi[b]))
        ds_.append(pltpu.make_async_copy(
            w_ref.at[pl.ds(base, S2_CH)], wv.at[b], semi[b]))
        return ds_

    def gather_descs(c, b):
        return [pltpu.make_async_copy(table_ref.at[srcv.at[_NJ * b + j]],
                                      rows.at[b, pl.ds(j * 128, 128)],
                                      semr[b])
                for j in range(_NJ)]

    def fire(descs):
        for d_ in descs:
            d_.start()

    def drain(descs):
        for d_ in descs:
            d_.wait()

    def process(c, b, fire_gnext, fire_inext):
        if fire_gnext:
            drain(idx_descs(c + 1, 1 - b))
            fire(gather_descs(c + 1, 1 - b))
        drain(gather_descs(c, b))

        def mul(i, c2):
            w16 = wv[b, pl.ds(i * 16, 16)]
            for l in range(16):
                r = i * 16 + l
                rows[b, r, pl.ds(0, 16)] = rows[b, r, pl.ds(0, 16)] * w16[l]
                rows[b, r, pl.ds(16, 16)] = rows[b, r, pl.ds(16, 16)] * w16[l]
            return c2
        lax.fori_loop(0, S2_CH // 16, mul, 0)
        for j in range(_NJ):
            pltpu.sync_copy(rows.at[b, pl.ds(j * 128, 128)],
                            acc.at[dstv.at[_NJ * b + j]], add=True)
        if fire_inext:
            fire(idx_descs(c + 2, b))

    fire(idx_descs(0, 0))
    drain(idx_descs(0, 0))
    fire(gather_descs(0, 0))
    fire(idx_descs(1, 1))

    def body(cj, carry):
        c = cj * 2
        process(c, 0, True, True)
        process(c + 1, 1, True, True)
        return carry
    lax.fori_loop(0, _NCH // 2 - 1, body, 0)
    process(_NCH - 2, 0, True, False)
    process(_NCH - 1, 1, False, False)
    plsc.subcore_barrier()

    for k in range(NSTR // SZB):
        pltpu.sync_copy(acc.at[pl.ds(sub * NSTR + k * SZB, SZB)],
                        rows.at[0, pl.ds(0, SZB)])
        pltpu.sync_copy(rows.at[0, pl.ds(0, SZB)],
                        out_ref.at[core, pl.ds(sub * NSTR + k * SZB, SZB)])


@functools.cache
def _build_s2():
    return pl.kernel(
        _s2_body,
        out_type=jax.ShapeDtypeStruct((NC, NPAD, HID), jnp.float32),
        mesh=_mesh(),
        compiler_params=pltpu.CompilerParams(use_tc_tiling_on_sc=False),
        scratch_types=[
            pltpu.VMEM((2 * _NJ, 128), jnp.int32),
            pltpu.VMEM((2 * _NJ, 128), jnp.int32),
            pltpu.VMEM((2, S2_CH), jnp.float32),
            pltpu.VMEM((2, S2_CH, HID), jnp.float32),
            pltpu.SemaphoreType.DMA,
            pltpu.SemaphoreType.DMA,
            pltpu.SemaphoreType.DMA,
            pltpu.SemaphoreType.DMA,
            pltpu.VMEM_SHARED((NPAD, HID), jnp.float32),
        ])


def _s2(*args):
    return _build_s2()(*args)


_PPT = PPAD // NW


def _s3_body(x2_ref, batch_ref, out_ref, batchv, rows, acc):
    core = lax.axis_index("c")
    sub = lax.axis_index("s")
    wid = sub * NC + core

    def zrow(i, carry):
        rows[i, pl.ds(0, 16)] = jnp.zeros((16,), jnp.float32)
        rows[i, pl.ds(16, 16)] = jnp.zeros((16,), jnp.float32)
        return carry
    lax.fori_loop(0, GSTR, zrow, 0)
    pltpu.sync_copy(rows.at[pl.ds(0, GSTR)], acc.at[pl.ds(sub * GSTR, GSTR)])
    plsc.subcore_barrier()

    pltpu.sync_copy(x2_ref.at[pl.ds(wid * _PPT, _PPT)], rows)
    for j in range(_PPT // 128):
        pltpu.sync_copy(batch_ref.at[pl.ds(wid * _PPT + j * 128, 128)],
                        batchv.at[j])
    for j in range(_PPT // 128):
        pltpu.sync_copy(rows.at[pl.ds(j * 128, 128)],
                        acc.at[batchv.at[j]], add=True)
    plsc.subcore_barrier()

    pltpu.sync_copy(acc.at[pl.ds(sub * GSTR, GSTR)], rows.at[pl.ds(0, GSTR)])
    pltpu.sync_copy(rows.at[pl.ds(0, GSTR)],
                    out_ref.at[core, pl.ds(sub * GSTR, GSTR)])


@functools.cache
def _build_s3():
    return pl.kernel(
        _s3_body,
        out_type=jax.ShapeDtypeStruct((NC, GPAD, OUT), jnp.float32),
        mesh=_mesh(),
        compiler_params=pltpu.CompilerParams(use_tc_tiling_on_sc=False),
        scratch_types=[
            pltpu.VMEM((_PPT // 128, 128), jnp.int32),
            pltpu.VMEM((_PPT, OUT), jnp.float32),
            pltpu.VMEM_SHARED((GPAD, OUT), jnp.float32),
        ])


def _s3(*args):
    return _build_s3()(*args)


def kernel(x, edge_index, edge_attr, batch, Wih_s, Whh_s, bih_s, bhh_s,
           Watt_s, Wfc_s, bfc_s, Wih_n, Whh_n, bih_n, bhh_n, Watt_n, Wfc_n,
           bfc_n, Wg1, bg1, Wg2, bg2, Wls, bls):
    f32 = jnp.float32
    x = x.astype(f32)

    xT_node = jnp.concatenate(
        [x.T, jnp.zeros((LAG, NPAD - N), f32)], 1).reshape(LAG, 1, NPAD)
    xsT = x.reshape(NG, 3, LAG).transpose(2, 1, 0)
    xsT = jnp.concatenate([xsT, jnp.zeros((LAG, 3, GPAD - NG), f32)], 2)
    src = edge_index[0].astype(jnp.int32)
    dst = edge_index[1].astype(jnp.int32)
    epad = E_PAD - E
    src_p = jnp.concatenate([src, jnp.zeros((epad,), jnp.int32)])
    dst_p = jnp.concatenate([dst, jnp.zeros((epad,), jnp.int32)])
    dst2 = dst_p.reshape(-1, 128)
    w_p = jnp.concatenate([edge_attr.astype(f32), jnp.zeros((epad,), f32)])
    batch1 = jnp.concatenate(
        [batch.astype(jnp.int32), jnp.full((PPAD - N,), NG, jnp.int32)])

    b_n = (bih_n + bhh_n).reshape(4 * HID, 1)
    b_s = (bih_s + bhh_s).reshape(4 * HID, 1)

    emb = _attlstm(xT_node, Wih_n, Whh_n, b_n,
                   Watt_n, Wfc_n, bfc_n.reshape(OUT, 1))

    src_e, dst_e, w_e = _s0(src_p, dst_p, w_p)
    degp, cntp = _s1(dst2, w_p.reshape(-1, 128), batch1)
    degp = degp.reshape(NC, NPAD)
    cntp = cntp.reshape(NC, GPAD)

    dis, h1s = _g0(degp.T, emb, Wg1)
    agg1p = _s2(h1s, src_e, dst_e, w_e)
    h2s = _g1(agg1p, dis, h1s, bg1.reshape(1, HID), Wg2)

    xsT_b, _ = lax.optimization_barrier((xsT, h2s))
    x1s = _attlstm(xsT_b, Wih_s, Whh_s, b_s,
                   Watt_s, Wfc_s, bfc_s.reshape(OUT, 1))

    agg2p = _s2(h2s, src_e, dst_e, w_e)
    x2 = _g2(agg2p, dis, h2s, bg2.reshape(1, OUT))

    x2_pool = jnp.concatenate([x2, jnp.zeros((PPAD - NPAD, OUT), f32)], 0)
    ssump = _s3(x2_pool, batch1)
    pred = _g3(ssump, cntp.T, x1s, Wls[:OUT].astype(f32),
               Wls[OUT:].astype(f32), bls.reshape(1, 1))
    return pred[:NG, 0]


# speedup vs baseline: 22.6775x; 1.0005x over previous
import functools

import jax
import jax.numpy as jnp
from jax import lax
from jax.experimental import pallas as pl
from jax.experimental.pallas import tpu as pltpu
from jax.experimental.pallas import tpu_sc as plsc

N = 50001
E = 1600032
NG = 16667
LAG = 30
HID = 32
OUT = 32

NC = 2
NS = 16
NW = NC * NS

NPAD = 50176
GPAD = 16896
PPAD = 53248
E_PAD = 1638400
EPT = E_PAD // NW

R_LSTM = 512
S1_CH = 1024
S2_CH = 256
SZB = 224
NSTR = NPAD // NS
GSTR = GPAD // NS

@functools.cache
def _mesh():
    return plsc.VectorSubcoreMesh(
        core_axis_name="c", subcore_axis_name="s",
        num_cores=NC, num_subcores=NS)


def _attlstm_body(x_ref, wih_ref, whh_ref, b_ref, watt_ref, wfc_ref,
                  bfc_ref, out_ref, hs_ref):
    R = x_ref.shape[2]
    D = x_ref.shape[1]
    whh = whh_ref[...]
    b = b_ref[...]
    hT = jnp.zeros((HID, R), jnp.float32)
    cT = jnp.zeros((HID, R), jnp.float32)
    scores = []
    for t in range(LAG):
        xtT = x_ref[t]
        if D == 1:
            gx = wih_ref[...] * xtT
        else:
            gx = jnp.dot(wih_ref[...], xtT,
                         preferred_element_type=jnp.float32)
        g = gx + jnp.dot(whh, hT, preferred_element_type=jnp.float32) + b
        i = jax.nn.sigmoid(g[0:HID])
        f = jax.nn.sigmoid(g[HID:2 * HID])
        gg = jnp.tanh(g[2 * HID:3 * HID])
        o = jax.nn.sigmoid(g[3 * HID:4 * HID])
        cT = f * cT + i * gg
        hT = o * jnp.tanh(cT)
        hs_ref[t] = hT
        scores.append(jnp.dot(watt_ref[t:t + 1, :], hT,
                              preferred_element_type=jnp.float32))
    s = jnp.concatenate(scores, axis=0)
    m = jnp.max(s, axis=0, keepdims=True)
    e = jnp.exp(s - m)
    a = e / jnp.sum(e, axis=0, keepdims=True)
    attT = jnp.zeros((HID, R), jnp.float32)
    for t in range(LAG):
        attT = attT + a[t:t + 1, :] * hs_ref[t]
    outT = (jnp.dot(wfc_ref[...], attT, preferred_element_type=jnp.float32)
            + bfc_ref[...])
    out_ref[...] = jnp.maximum(outT, 0.0)


def _attlstm(xpT, wih, whh, b2, watt, wfc, bfc2):
    _, D, B = xpT.shape
    grid = B // R_LSTM
    outT = pl.pallas_call(
        _attlstm_body,
        grid=(grid,),
        in_specs=[
            pl.BlockSpec((LAG, D, R_LSTM), lambda i: (0, 0, i)),
            pl.BlockSpec((4 * HID, D), lambda i: (0, 0)),
            pl.BlockSpec((4 * HID, HID), lambda i: (0, 0)),
            pl.BlockSpec((4 * HID, 1), lambda i: (0, 0)),
            pl.BlockSpec((LAG, HID), lambda i: (0, 0)),
            pl.BlockSpec((OUT, HID), lambda i: (0, 0)),
            pl.BlockSpec((OUT, 1), lambda i: (0, 0)),
        ],
        out_specs=pl.BlockSpec((OUT, R_LSTM), lambda i: (0, i)),
        out_shape=jax.ShapeDtypeStruct((OUT, B), jnp.float32),
        scratch_shapes=[pltpu.VMEM((LAG, HID, R_LSTM), jnp.float32)],
    )(xpT, wih, whh, b2, watt, wfc, bfc2)
    return outT.T


_RG = 3584


def _g0_body(degp_ref, emb_ref, wg1_ref, dis_ref, h1s_ref):
    deg = degp_ref[:, 0:1] + degp_ref[:, 1:2] + 1.0
    dis = lax.rsqrt(deg)
    h1 = jnp.dot(emb_ref[...], wg1_ref[...], preferred_element_type=jnp.float32)
    dis_ref[...] = dis
    h1s_ref[...] = dis * h1


def _g0(degpT, emb, wg1):
    grid = NPAD // _RG
    return pl.pallas_call(
        _g0_body,
        grid=(grid,),
        in_specs=[
            pl.BlockSpec((_RG, NC), lambda i: (i, 0)),
            pl.BlockSpec((_RG, HID), lambda i: (i, 0)),
            pl.BlockSpec((HID, HID), lambda i: (0, 0)),
        ],
        out_specs=[
            pl.BlockSpec((_RG, 1), lambda i: (i, 0)),
            pl.BlockSpec((_RG, HID), lambda i: (i, 0)),
        ],
        out_shape=[
            jax.ShapeDtypeStruct((NPAD, 1), jnp.float32),
            jax.ShapeDtypeStruct((NPAD, HID), jnp.float32),
        ],
    )(degpT, emb, wg1)


def _g1_body(aggp_ref, dis_ref, h1s_ref, bg1_ref, wg2_ref, h2s_ref):
    dis = dis_ref[...]
    x2a = dis * (aggp_ref[0] + aggp_ref[1] + h1s_ref[...]) + bg1_ref[...]
    h2 = jnp.dot(x2a, wg2_ref[...], preferred_element_type=jnp.float32)
    h2s_ref[...] = dis * h2


def _g1(aggp, dis, h1s, bg1, wg2):
    grid = NPAD // _RG
    return pl.pallas_call(
        _g1_body,
        grid=(grid,),
        in_specs=[
            pl.BlockSpec((NC, _RG, HID), lambda i: (0, i, 0)),
            pl.BlockSpec((_RG, 1), lambda i: (i, 0)),
            pl.BlockSpec((_RG, HID), lambda i: (i, 0)),
            pl.BlockSpec((1, HID), lambda i: (0, 0)),
            pl.BlockSpec((HID, OUT), lambda i: (0, 0)),
        ],
        out_specs=pl.BlockSpec((_RG, OUT), lambda i: (i, 0)),
        out_shape=jax.ShapeDtypeStruct((NPAD, OUT), jnp.float32),
    )(aggp, dis, h1s, bg1, wg2)


def _g2_body(aggp_ref, dis_ref, h2s_ref, bg2_ref, x2_ref):
    dis = dis_ref[...]
    x2_ref[...] = dis * (aggp_ref[0] + aggp_ref[1] + h2s_ref[...]) + bg2_ref[...]


def _g2(aggp, dis, h2s, bg2):
    grid = NPAD // _RG
    return pl.pallas_call(
        _g2_body,
        grid=(grid,),
        in_specs=[
            pl.BlockSpec((NC, _RG, OUT), lambda i: (0, i, 0)),
            pl.BlockSpec((_RG, 1), lambda i: (i, 0)),
            pl.BlockSpec((_RG, OUT), lambda i: (i, 0)),
            pl.BlockSpec((1, OUT), lambda i: (0, 0)),
        ],
        out_specs=pl.BlockSpec((_RG, OUT), lambda i: (i, 0)),
        out_shape=jax.ShapeDtypeStruct((NPAD, OUT), jnp.float32),
    )(aggp, dis, h2s, bg2)


_RG3 = 2112


def _g3_body(ssump_ref, cntp_ref, x1s_ref, wa_ref, wb_ref, bls_ref, out_ref):
    cnt = jnp.maximum(cntp_ref[:, 0:1] + cntp_ref[:, 1:2], 1.0)
    x2n = (ssump_ref[0] + ssump_ref[1]) / cnt
    pred = (jnp.dot(x1s_ref[...], wa_ref[...], preferred_element_type=jnp.float32)
            + jnp.dot(x2n, wb_ref[...], preferred_element_type=jnp.float32)
            + bls_ref[...])
    out_ref[...] = jnp.maximum(pred, 0.0)


def _g3(ssump, cntpT, x1s, wa, wb, bls2):
    grid = GPAD // _RG3
    return pl.pallas_call(
        _g3_body,
        grid=(grid,),
        in_specs=[
            pl.BlockSpec((NC, _RG3, OUT), lambda i: (0, i, 0)),
            pl.BlockSpec((_RG3, NC), lambda i: (i, 0)),
            pl.BlockSpec((_RG3, OUT), lambda i: (i, 0)),
            pl.BlockSpec((OUT, 1), lambda i: (0, 0)),
            pl.BlockSpec((OUT, 1), lambda i: (0, 0)),
            pl.BlockSpec((1, 1), lambda i: (0, 0)),
        ],
        out_specs=pl.BlockSpec((_RG3, 1), lambda i: (i, 0)),
        out_shape=jax.ShapeDtypeStruct((GPAD, 1), jnp.float32),
    )(ssump, cntpT, x1s, wa, wb, bls2)


_PCH = 6400


def _s0_body(src_ref, dst_ref, w_ref, srco_ref, dsto_ref, wo_ref, bi, bf):
    core = lax.axis_index("c")
    sub = lax.axis_index("s")
    wid = sub * NC + core

    def chunk(ci, carry):
        base = wid * EPT + ci * _PCH
        for a, o, b in ((src_ref, srco_ref, bi), (dst_ref, dsto_ref, bi),
                        (w_ref, wo_ref, bf)):
            pltpu.sync_copy(a.at[pl.ds(base, _PCH)], b)
            pltpu.sync_copy(b, o.at[pl.ds(base, _PCH)])
        return carry
    lax.fori_loop(0, EPT // _PCH, chunk, 0)


@functools.cache
def _build_s0():
    return pl.kernel(
        _s0_body,
        out_type=(jax.ShapeDtypeStruct((E_PAD,), jnp.int32),
                  jax.ShapeDtypeStruct((E_PAD,), jnp.int32),
                  jax.ShapeDtypeStruct((E_PAD,), jnp.float32)),
        mesh=_mesh(),
        compiler_params=pltpu.CompilerParams(use_tc_tiling_on_sc=False),
        scratch_types=[pltpu.VMEM((_PCH,), jnp.int32),
                       pltpu.VMEM((_PCH,), jnp.float32)])


def _s0(*args):
    return _build_s0()(*args)


def _memset(ref, n, val):
    def step(i, carry):
        ref[pl.ds(i * 16, 16)] = jnp.full((16,), val, jnp.float32)
        return carry
    lax.fori_loop(0, n // 16, step, 0)


def _s1_body(dst_ref, w_ref, batch_ref, degout_ref, cntout_ref,
             dstv, wv, batchv, zbuf, onesv, dacc, cacc):
    core = lax.axis_index("c")
    sub = lax.axis_index("s")
    wid = sub * NC + core
    _memset(zbuf, NSTR, 0.0)
    _memset(onesv, 128, 1.0)
    pltpu.sync_copy(zbuf.at[pl.ds(0, NSTR)], dacc.at[pl.ds(sub * NSTR, NSTR)])
    pltpu.sync_copy(zbuf.at[pl.ds(0, GSTR)], cacc.at[pl.ds(sub * GSTR, GSTR)])
    plsc.subcore_barrier()

    def echunk(ci, carry):
        row0 = wid * (EPT // 128) + ci * (S1_CH // 128)
        pltpu.sync_copy(dst_ref.at[pl.ds(row0, S1_CH // 128)], dstv)
        pltpu.sync_copy(w_ref.at[pl.ds(row0, S1_CH // 128)], wv)
        for j in range(S1_CH // 128):
            pltpu.sync_copy(wv.at[j], dacc.at[dstv.at[j]], add=True)
        return carry
    lax.fori_loop(0, EPT // S1_CH, echunk, 0)

    for j in range(PPAD // NW // 128):
        pltpu.sync_copy(batch_ref.at[pl.ds(wid * (PPAD // NW) + j * 128, 128)],
                        batchv.at[j])
    for j in range(PPAD // NW // 128):
        pltpu.sync_copy(onesv, cacc.at[batchv.at[j]], add=True)
    plsc.subcore_barrier()

    pltpu.sync_copy(dacc.at[pl.ds(sub * NSTR, NSTR)], zbuf.at[pl.ds(0, NSTR)])
    pltpu.sync_copy(zbuf.at[pl.ds(0, NSTR)],
                    degout_ref.at[pl.ds(core * NPAD + sub * NSTR, NSTR)])
    pltpu.sync_copy(cacc.at[pl.ds(sub * GSTR, GSTR)], zbuf.at[pl.ds(0, GSTR)])
    pltpu.sync_copy(zbuf.at[pl.ds(0, GSTR)],
                    cntout_ref.at[pl.ds(core * GPAD + sub * GSTR, GSTR)])


@functools.cache
def _build_s1():
    return pl.kernel(
        _s1_body,
        out_type=(jax.ShapeDtypeStruct((NC * NPAD,), jnp.float32),
                  jax.ShapeDtypeStruct((NC * GPAD,), jnp.float32)),
        mesh=_mesh(),
        compiler_params=pltpu.CompilerParams(use_tc_tiling_on_sc=False),
        scratch_types=[
            pltpu.VMEM((S1_CH // 128, 128), jnp.int32),
            pltpu.VMEM((S1_CH // 128, 128), jnp.float32),
            pltpu.VMEM((PPAD // NW // 128, 128), jnp.int32),
            pltpu.VMEM((NSTR,), jnp.float32),
            pltpu.VMEM((128,), jnp.float32),
            pltpu.VMEM_SHARED((NPAD,), jnp.float32),
            pltpu.VMEM_SHARED((GPAD,), jnp.float32),
        ])


def _s1(*args):
    return _build_s1()(*args)


_NJ = S2_CH // 128
_NCH = EPT // S2_CH


def _s2_body(table_ref, src_ref, dst_ref, w_ref, out_ref,
             srcv, dstv, wv, rows, semr0, semr1, semi0, semi1, acc):
    core = lax.axis_index("c")
    sub = lax.axis_index("s")
    wid = sub * NC + core
    semr = (semr0, semr1)
    semi = (semi0, semi1)

    def zrow(i, carry):
        rows[0, i, pl.ds(0, 16)] = jnp.zeros((16,), jnp.float32)
        rows[0, i, pl.ds(16, 16)] = jnp.zeros((16,), jnp.float32)
        return carry
    lax.fori_loop(0, S2_CH, zrow, 0)
    for k in range(NSTR // SZB):
        pltpu.sync_copy(rows.at[0, pl.ds(0, SZB)],
                        acc.at[pl.ds(sub * NSTR + k * SZB, SZB)])
    plsc.subcore_barrier()

    def idx_descs(c, b):
        base = wid * EPT + c * S2_CH
        ds_ = []
        for j in range(_NJ):
            ds_.append(pltpu.make_async_copy(
                src_ref.at[pl.ds(base + j * 128, 128)],
                srcv.at[_NJ * b + j], semi[b]))
            ds_.append(pltpu.make_async_copy(
                dst_ref.at[pl.ds(base + j * 128, 128)],
                dstv.at[_NJ * b + j], semi[b]))
        ds_.append(pltpu.make_async_copy(
            w_ref.at[pl.ds(base, S2_CH)], wv.at[b], semi[b]))
        return ds_

    def gather_descs(c, b):
        return [pltpu.make_async_copy(table_ref.at[srcv.at[_NJ * b + j]],
                                      rows.at[b, pl.ds(j * 128, 128)],
                                      semr[b])
                for j in range(_NJ)]

    def fire(descs):
        for d_ in descs:
            d_.start()

    def drain(descs):
        for d_ in descs:
            d_.wait()

    def process(c, b, fire_gnext, fire_inext):
        if fire_gnext:
            drain(idx_descs(c + 1, 1 - b))
            fire(gather_descs(c + 1, 1 - b))
        drain(gather_descs(c, b))

        def mul(i, c2):
            w16 = wv[b, pl.ds(i * 16, 16)]
            for l in range(16):
                r = i * 16 + l
                rows[b, r, pl.ds(0, 16)] = rows[b, r, pl.ds(0, 16)] * w16[l]
                rows[b, r, pl.ds(16, 16)] = rows[b, r, pl.ds(16, 16)] * w16[l]
            return c2
        lax.fori_loop(0, S2_CH // 16, mul, 0)
        for j in range(_NJ):
            pltpu.sync_copy(rows.at[b, pl.ds(j * 128, 128)],
                            acc.at[dstv.at[_NJ * b + j]], add=True)
        if fire_inext:
            fire(idx_descs(c + 2, b))

    fire(idx_descs(0, 0))
    drain(idx_descs(0, 0))
    fire(gather_descs(0, 0))
    fire(idx_descs(1, 1))

    def body(cj, carry):
        c = cj * 2
        process(c, 0, True, True)
        process(c + 1, 1, True, True)
        return carry
    lax.fori_loop(0, _NCH // 2 - 1, body, 0)
    process(_NCH - 2, 0, True, False)
    process(_NCH - 1, 1, False, False)
    plsc.subcore_barrier()

    for k in range(NSTR // SZB):
        pltpu.sync_copy(acc.at[pl.ds(sub * NSTR + k * SZB, SZB)],
                        rows.at[0, pl.ds(0, SZB)])
        pltpu.sync_copy(rows.at[0, pl.ds(0, SZB)],
                        out_ref.at[core, pl.ds(sub * NSTR + k * SZB, SZB)])


@functools.cache
def _build_s2():
    return pl.kernel(
        _s2_body,
        out_type=jax.ShapeDtypeStruct((NC, NPAD, HID), jnp.float32),
        mesh=_mesh(),
        compiler_params=pltpu.CompilerParams(use_tc_tiling_on_sc=False),
        scratch_types=[
            pltpu.VMEM((2 * _NJ, 128), jnp.int32),
            pltpu.VMEM((2 * _NJ, 128), jnp.int32),
            pltpu.VMEM((2, S2_CH), jnp.float32),
            pltpu.VMEM((2, S2_CH, HID), jnp.float32),
            pltpu.SemaphoreType.DMA,
            pltpu.SemaphoreType.DMA,
            pltpu.SemaphoreType.DMA,
            pltpu.SemaphoreType.DMA,
            pltpu.VMEM_SHARED((NPAD, HID), jnp.float32),
        ])


def _s2(*args):
    return _build_s2()(*args)


_PPT = PPAD // NW


def _s3_body(x2_ref, batch_ref, out_ref, batchv, rows, acc):
    core = lax.axis_index("c")
    sub = lax.axis_index("s")
    wid = sub * NC + core

    def zrow(i, carry):
        rows[i, pl.ds(0, 16)] = jnp.zeros((16,), jnp.float32)
        rows[i, pl.ds(16, 16)] = jnp.zeros((16,), jnp.float32)
        return carry
    lax.fori_loop(0, GSTR, zrow, 0)
    pltpu.sync_copy(rows.at[pl.ds(0, GSTR)], acc.at[pl.ds(sub * GSTR, GSTR)])
    plsc.subcore_barrier()

    pltpu.sync_copy(x2_ref.at[pl.ds(wid * _PPT, _PPT)], rows)
    for j in range(_PPT // 128):
        pltpu.sync_copy(batch_ref.at[pl.ds(wid * _PPT + j * 128, 128)],
                        batchv.at[j])
    for j in range(_PPT // 128):
        pltpu.sync_copy(rows.at[pl.ds(j * 128, 128)],
                        acc.at[batchv.at[j]], add=True)
    plsc.subcore_barrier()

    pltpu.sync_copy(acc.at[pl.ds(sub * GSTR, GSTR)], rows.at[pl.ds(0, GSTR)])
    pltpu.sync_copy(rows.at[pl.ds(0, GSTR)],
                    out_ref.at[core, pl.ds(sub * GSTR, GSTR)])


@functools.cache
def _build_s3():
    return pl.kernel(
        _s3_body,
        out_type=jax.ShapeDtypeStruct((NC, GPAD, OUT), jnp.float32),
        mesh=_mesh(),
        compiler_params=pltpu.CompilerParams(use_tc_tiling_on_sc=False),
        scratch_types=[
            pltpu.VMEM((_PPT // 128, 128), jnp.int32),
            pltpu.VMEM((_PPT, OUT), jnp.float32),
            pltpu.VMEM_SHARED((GPAD, OUT), jnp.float32),
        ])


def _s3(*args):
    return _build_s3()(*args)


def kernel(x, edge_index, edge_attr, batch, Wih_s, Whh_s, bih_s, bhh_s,
           Watt_s, Wfc_s, bfc_s, Wih_n, Whh_n, bih_n, bhh_n, Watt_n, Wfc_n,
           bfc_n, Wg1, bg1, Wg2, bg2, Wls, bls):
    f32 = jnp.float32
    x = x.astype(f32)

    xT_node = jnp.concatenate(
        [x.T, jnp.zeros((LAG, NPAD - N), f32)], 1).reshape(LAG, 1, NPAD)
    xsT = x.reshape(NG, 3, LAG).transpose(2, 1, 0)
    xsT = jnp.concatenate([xsT, jnp.zeros((LAG, 3, GPAD - NG), f32)], 2)
    src = edge_index[0].astype(jnp.int32)
    dst = edge_index[1].astype(jnp.int32)
    epad = E_PAD - E
    src_p = jnp.concatenate([src, jnp.zeros((epad,), jnp.int32)])
    dst_p = jnp.concatenate([dst, jnp.zeros((epad,), jnp.int32)])
    dst2 = dst_p.reshape(-1, 128)
    w_p = jnp.concatenate([edge_attr.astype(f32), jnp.zeros((epad,), f32)])
    batch1 = jnp.concatenate(
        [batch.astype(jnp.int32), jnp.full((PPAD - N,), NG, jnp.int32)])

    b_n = (bih_n + bhh_n).reshape(4 * HID, 1)
    b_s = (bih_s + bhh_s).reshape(4 * HID, 1)

    emb = _attlstm(xT_node, Wih_n, Whh_n, b_n,
                   Watt_n, Wfc_n, bfc_n.reshape(OUT, 1))

    src_e, dst_e, w_e = _s0(src_p, dst_p, w_p)
    degp, cntp = _s1(dst2, w_p.reshape(-1, 128), batch1)
    degp = degp.reshape(NC, NPAD)
    cntp = cntp.reshape(NC, GPAD)

    dis, h1s = _g0(degp.T, emb, Wg1)
    agg1p = _s2(h1s, src_e, dst_e, w_e)

    xsT_b, _ = lax.optimization_barrier((xsT, h1s))
    x1s = _attlstm(xsT_b, Wih_s, Whh_s, b_s,
                   Watt_s, Wfc_s, bfc_s.reshape(OUT, 1))

    h2s = _g1(agg1p, dis, h1s, bg1.reshape(1, HID), Wg2)

    agg2p = _s2(h2s, src_e, dst_e, w_e)
    x2 = _g2(agg2p, dis, h2s, bg2.reshape(1, OUT))

    x2_pool = jnp.concatenate([x2, jnp.zeros((PPAD - NPAD, OUT), f32)], 0)
    ssump = _s3(x2_pool, batch1)
    pred = _g3(ssump, cntp.T, x1s, Wls[:OUT].astype(f32),
               Wls[OUT:].astype(f32), bls.reshape(1, 1))
    return pred[:NG, 0]


# speedup vs baseline: 24.5728x; 1.0836x over previous
import functools

import jax
import jax.numpy as jnp
from jax import lax
from jax.experimental import pallas as pl
from jax.experimental.pallas import tpu as pltpu
from jax.experimental.pallas import tpu_sc as plsc

N = 50001
E = 1600032
NG = 16667
LAG = 30
HID = 32
OUT = 32

NC = 2
NS = 16
NW = NC * NS

NPAD = 50176
GPAD = 16896
PPAD = 53248
E_PAD = 1638400
EPT = E_PAD // NW

R_LSTM = 512
S1_CH = 1024
S2_CH = 256
SZB = 224
NSTR = NPAD // NS
GSTR = GPAD // NS

@functools.cache
def _mesh():
    return plsc.VectorSubcoreMesh(
        core_axis_name="c", subcore_axis_name="s",
        num_cores=NC, num_subcores=NS)


def _attlstm_body(x_ref, wih_ref, whh_ref, b_ref, watt_ref, wfc_ref,
                  bfc_ref, out_ref, hs_ref):
    R = x_ref.shape[2]
    D = x_ref.shape[1]
    whh = whh_ref[...]
    b = b_ref[...]
    hT = jnp.zeros((HID, R), jnp.float32)
    cT = jnp.zeros((HID, R), jnp.float32)
    scores = []
    for t in range(LAG):
        xtT = x_ref[t]
        if D == 1:
            gx = wih_ref[...] * xtT
        else:
            gx = jnp.dot(wih_ref[...], xtT,
                         preferred_element_type=jnp.float32)
        g = gx + jnp.dot(whh, hT, preferred_element_type=jnp.float32) + b
        i = jax.nn.sigmoid(g[0:HID])
        f = jax.nn.sigmoid(g[HID:2 * HID])
        gg = jnp.tanh(g[2 * HID:3 * HID])
        o = jax.nn.sigmoid(g[3 * HID:4 * HID])
        cT = f * cT + i * gg
        hT = o * jnp.tanh(cT)
        hs_ref[t] = hT
        scores.append(jnp.dot(watt_ref[t:t + 1, :], hT,
                              preferred_element_type=jnp.float32))
    s = jnp.concatenate(scores, axis=0)
    m = jnp.max(s, axis=0, keepdims=True)
    e = jnp.exp(s - m)
    a = e / jnp.sum(e, axis=0, keepdims=True)
    attT = jnp.zeros((HID, R), jnp.float32)
    for t in range(LAG):
        attT = attT + a[t:t + 1, :] * hs_ref[t]
    outT = (jnp.dot(wfc_ref[...], attT, preferred_element_type=jnp.float32)
            + bfc_ref[...])
    out_ref[...] = jnp.maximum(outT, 0.0)


def _attlstm(xpT, wih, whh, b2, watt, wfc, bfc2):
    _, D, B = xpT.shape
    grid = B // R_LSTM
    outT = pl.pallas_call(
        _attlstm_body,
        grid=(grid,),
        in_specs=[
            pl.BlockSpec((LAG, D, R_LSTM), lambda i: (0, 0, i)),
            pl.BlockSpec((4 * HID, D), lambda i: (0, 0)),
            pl.BlockSpec((4 * HID, HID), lambda i: (0, 0)),
            pl.BlockSpec((4 * HID, 1), lambda i: (0, 0)),
            pl.BlockSpec((LAG, HID), lambda i: (0, 0)),
            pl.BlockSpec((OUT, HID), lambda i: (0, 0)),
            pl.BlockSpec((OUT, 1), lambda i: (0, 0)),
        ],
        out_specs=pl.BlockSpec((OUT, R_LSTM), lambda i: (0, i)),
        out_shape=jax.ShapeDtypeStruct((OUT, B), jnp.float32),
        scratch_shapes=[pltpu.VMEM((LAG, HID, R_LSTM), jnp.float32)],
    )(xpT, wih, whh, b2, watt, wfc, bfc2)
    return outT.T


_RG = 3584


def _pack_bf16(hs):
    a = jax.lax.convert_element_type(hs[:, 0:HID // 2], jnp.bfloat16)
    b = jax.lax.convert_element_type(hs[:, HID // 2:], jnp.bfloat16)
    au = jax.lax.bitcast_convert_type(a, jnp.uint16).astype(jnp.uint32)
    bu = jax.lax.bitcast_convert_type(b, jnp.uint16).astype(jnp.uint32)
    return jax.lax.bitcast_convert_type(au | (bu << 16), jnp.int32)


def _g0_body(degp_ref, emb_ref, wg1_ref, dis_ref, h1s_ref, t1_ref):
    deg = degp_ref[:, 0:1] + degp_ref[:, 1:2] + 1.0
    dis = lax.rsqrt(deg)
    h1 = jnp.dot(emb_ref[...], wg1_ref[...], preferred_element_type=jnp.float32)
    dis_ref[...] = dis
    h1s = dis * h1
    h1s_ref[...] = h1s
    t1_ref[...] = _pack_bf16(h1s)


def _g0(degpT, emb, wg1):
    grid = NPAD // _RG
    return pl.pallas_call(
        _g0_body,
        grid=(grid,),
        in_specs=[
            pl.BlockSpec((_RG, NC), lambda i: (i, 0)),
            pl.BlockSpec((_RG, HID), lambda i: (i, 0)),
            pl.BlockSpec((HID, HID), lambda i: (0, 0)),
        ],
        out_specs=[
            pl.BlockSpec((_RG, 1), lambda i: (i, 0)),
            pl.BlockSpec((_RG, HID), lambda i: (i, 0)),
            pl.BlockSpec((_RG, HID // 2), lambda i: (i, 0)),
        ],
        out_shape=[
            jax.ShapeDtypeStruct((NPAD, 1), jnp.float32),
            jax.ShapeDtypeStruct((NPAD, HID), jnp.float32),
            jax.ShapeDtypeStruct((NPAD, HID // 2), jnp.int32),
        ],
    )(degpT, emb, wg1)


def _g1_body(aggp_ref, dis_ref, h1s_ref, bg1_ref, wg2_ref, h2s_ref, t2_ref):
    dis = dis_ref[...]
    x2a = dis * (aggp_ref[0] + aggp_ref[1] + h1s_ref[...]) + bg1_ref[...]
    h2 = jnp.dot(x2a, wg2_ref[...], preferred_element_type=jnp.float32)
    h2s = dis * h2
    h2s_ref[...] = h2s
    t2_ref[...] = _pack_bf16(h2s)


def _g1(aggp, dis, h1s, bg1, wg2):
    grid = NPAD // _RG
    return pl.pallas_call(
        _g1_body,
        grid=(grid,),
        in_specs=[
            pl.BlockSpec((NC, _RG, HID), lambda i: (0, i, 0)),
            pl.BlockSpec((_RG, 1), lambda i: (i, 0)),
            pl.BlockSpec((_RG, HID), lambda i: (i, 0)),
            pl.BlockSpec((1, HID), lambda i: (0, 0)),
            pl.BlockSpec((HID, OUT), lambda i: (0, 0)),
        ],
        out_specs=[
            pl.BlockSpec((_RG, OUT), lambda i: (i, 0)),
            pl.BlockSpec((_RG, HID // 2), lambda i: (i, 0)),
        ],
        out_shape=[
            jax.ShapeDtypeStruct((NPAD, OUT), jnp.float32),
            jax.ShapeDtypeStruct((NPAD, HID // 2), jnp.int32),
        ],
    )(aggp, dis, h1s, bg1, wg2)


def _g2_body(aggp_ref, dis_ref, h2s_ref, bg2_ref, x2_ref):
    dis = dis_ref[...]
    x2_ref[...] = dis * (aggp_ref[0] + aggp_ref[1] + h2s_ref[...]) + bg2_ref[...]


def _g2(aggp, dis, h2s, bg2):
    grid = NPAD // _RG
    return pl.pallas_call(
        _g2_body,
        grid=(grid,),
        in_specs=[
            pl.BlockSpec((NC, _RG, OUT), lambda i: (0, i, 0)),
            pl.BlockSpec((_RG, 1), lambda i: (i, 0)),
            pl.BlockSpec((_RG, OUT), lambda i: (i, 0)),
            pl.BlockSpec((1, OUT), lambda i: (0, 0)),
        ],
        out_specs=pl.BlockSpec((_RG, OUT), lambda i: (i, 0)),
        out_shape=jax.ShapeDtypeStruct((NPAD, OUT), jnp.float32),
    )(aggp, dis, h2s, bg2)


_RG3 = 2112


def _g3_body(ssump_ref, cntp_ref, x1s_ref, wa_ref, wb_ref, bls_ref, out_ref):
    cnt = jnp.maximum(cntp_ref[:, 0:1] + cntp_ref[:, 1:2], 1.0)
    x2n = (ssump_ref[0] + ssump_ref[1]) / cnt
    pred = (jnp.dot(x1s_ref[...], wa_ref[...], preferred_element_type=jnp.float32)
            + jnp.dot(x2n, wb_ref[...], preferred_element_type=jnp.float32)
            + bls_ref[...])
    out_ref[...] = jnp.maximum(pred, 0.0)


def _g3(ssump, cntpT, x1s, wa, wb, bls2):
    grid = GPAD // _RG3
    return pl.pallas_call(
        _g3_body,
        grid=(grid,),
        in_specs=[
            pl.BlockSpec((NC, _RG3, OUT), lambda i: (0, i, 0)),
            pl.BlockSpec((_RG3, NC), lambda i: (i, 0)),
            pl.BlockSpec((_RG3, OUT), lambda i: (i, 0)),
            pl.BlockSpec((OUT, 1), lambda i: (0, 0)),
            pl.BlockSpec((OUT, 1), lambda i: (0, 0)),
            pl.BlockSpec((1, 1), lambda i: (0, 0)),
        ],
        out_specs=pl.BlockSpec((_RG3, 1), lambda i: (i, 0)),
        out_shape=jax.ShapeDtypeStruct((GPAD, 1), jnp.float32),
    )(ssump, cntpT, x1s, wa, wb, bls2)


_PCH = 6400


def _s0_body(src_ref, dst_ref, w_ref, srco_ref, dsto_ref, wo_ref, bi, bf):
    core = lax.axis_index("c")
    sub = lax.axis_index("s")
    wid = sub * NC + core

    def chunk(ci, carry):
        base = wid * EPT + ci * _PCH
        for a, o, b in ((src_ref, srco_ref, bi), (dst_ref, dsto_ref, bi),
                        (w_ref, wo_ref, bf)):
            pltpu.sync_copy(a.at[pl.ds(base, _PCH)], b)
            pltpu.sync_copy(b, o.at[pl.ds(base, _PCH)])
        return carry
    lax.fori_loop(0, EPT // _PCH, chunk, 0)


@functools.cache
def _build_s0():
    return pl.kernel(
        _s0_body,
        out_type=(jax.ShapeDtypeStruct((E_PAD,), jnp.int32),
                  jax.ShapeDtypeStruct((E_PAD,), jnp.int32),
                  jax.ShapeDtypeStruct((E_PAD,), jnp.float32)),
        mesh=_mesh(),
        compiler_params=pltpu.CompilerParams(use_tc_tiling_on_sc=False),
        scratch_types=[pltpu.VMEM((_PCH,), jnp.int32),
                       pltpu.VMEM((_PCH,), jnp.float32)])


def _s0(*args):
    return _build_s0()(*args)


def _memset(ref, n, val):
    def step(i, carry):
        ref[pl.ds(i * 16, 16)] = jnp.full((16,), val, jnp.float32)
        return carry
    lax.fori_loop(0, n // 16, step, 0)


def _s1_body(dst_ref, w_ref, batch_ref, degout_ref, cntout_ref,
             dstv, wv, batchv, zbuf, onesv, dacc, cacc):
    core = lax.axis_index("c")
    sub = lax.axis_index("s")
    wid = sub * NC + core
    _memset(zbuf, NSTR, 0.0)
    _memset(onesv, 128, 1.0)
    pltpu.sync_copy(zbuf.at[pl.ds(0, NSTR)], dacc.at[pl.ds(sub * NSTR, NSTR)])
    pltpu.sync_copy(zbuf.at[pl.ds(0, GSTR)], cacc.at[pl.ds(sub * GSTR, GSTR)])
    plsc.subcore_barrier()

    def echunk(ci, carry):
        row0 = wid * (EPT // 128) + ci * (S1_CH // 128)
        pltpu.sync_copy(dst_ref.at[pl.ds(row0, S1_CH // 128)], dstv)
        pltpu.sync_copy(w_ref.at[pl.ds(row0, S1_CH // 128)], wv)
        for j in range(S1_CH // 128):
            pltpu.sync_copy(wv.at[j], dacc.at[dstv.at[j]], add=True)
        return carry
    lax.fori_loop(0, EPT // S1_CH, echunk, 0)

    for j in range(PPAD // NW // 128):
        pltpu.sync_copy(batch_ref.at[pl.ds(wid * (PPAD // NW) + j * 128, 128)],
                        batchv.at[j])
    for j in range(PPAD // NW // 128):
        pltpu.sync_copy(onesv, cacc.at[batchv.at[j]], add=True)
    plsc.subcore_barrier()

    pltpu.sync_copy(dacc.at[pl.ds(sub * NSTR, NSTR)], zbuf.at[pl.ds(0, NSTR)])
    pltpu.sync_copy(zbuf.at[pl.ds(0, NSTR)],
                    degout_ref.at[pl.ds(core * NPAD + sub * NSTR, NSTR)])
    pltpu.sync_copy(cacc.at[pl.ds(sub * GSTR, GSTR)], zbuf.at[pl.ds(0, GSTR)])
    pltpu.sync_copy(zbuf.at[pl.ds(0, GSTR)],
                    cntout_ref.at[pl.ds(core * GPAD + sub * GSTR, GSTR)])


@functools.cache
def _build_s1():
    return pl.kernel(
        _s1_body,
        out_type=(jax.ShapeDtypeStruct((NC * NPAD,), jnp.float32),
                  jax.ShapeDtypeStruct((NC * GPAD,), jnp.float32)),
        mesh=_mesh(),
        compiler_params=pltpu.CompilerParams(use_tc_tiling_on_sc=False),
        scratch_types=[
            pltpu.VMEM((S1_CH // 128, 128), jnp.int32),
            pltpu.VMEM((S1_CH // 128, 128), jnp.float32),
            pltpu.VMEM((PPAD // NW // 128, 128), jnp.int32),
            pltpu.VMEM((NSTR,), jnp.float32),
            pltpu.VMEM((128,), jnp.float32),
            pltpu.VMEM_SHARED((NPAD,), jnp.float32),
            pltpu.VMEM_SHARED((GPAD,), jnp.float32),
        ])


def _s1(*args):
    return _build_s1()(*args)


_NJ = S2_CH // 128
_NCH = EPT // S2_CH


def _s2_body(table_ref, src_ref, dst_ref, w_ref, out_ref,
             srcv, dstv, wv, rows, frows, semr0, semr1, semi0, semi1, acc):
    core = lax.axis_index("c")
    sub = lax.axis_index("s")
    wid = sub * NC + core
    semr = (semr0, semr1)
    semi = (semi0, semi1)

    def zrow(i, carry):
        frows[i, pl.ds(0, 16)] = jnp.zeros((16,), jnp.float32)
        frows[i, pl.ds(16, 16)] = jnp.zeros((16,), jnp.float32)
        return carry
    lax.fori_loop(0, S2_CH, zrow, 0)
    for k in range(NSTR // SZB):
        pltpu.sync_copy(frows.at[pl.ds(0, SZB)],
                        acc.at[pl.ds(sub * NSTR + k * SZB, SZB)])
    plsc.subcore_barrier()

    def idx_descs(c, b):
        base = wid * EPT + c * S2_CH
        ds_ = []
        for j in range(_NJ):
            ds_.append(pltpu.make_async_copy(
                src_ref.at[pl.ds(base + j * 128, 128)],
                srcv.at[_NJ * b + j], semi[b]))
            ds_.append(pltpu.make_async_copy(
                dst_ref.at[pl.ds(base + j * 128, 128)],
                dstv.at[_NJ * b + j], semi[b]))
        ds_.append(pltpu.make_async_copy(
            w_ref.at[pl.ds(base, S2_CH)], wv.at[b], semi[b]))
        return ds_

    def gather_descs(c, b):
        return [pltpu.make_async_copy(table_ref.at[srcv.at[_NJ * b + j]],
                                      rows.at[b, pl.ds(j * 128, 128)],
                                      semr[b])
                for j in range(_NJ)]

    def fire(descs):
        for d_ in descs:
            d_.start()

    def drain(descs):
        for d_ in descs:
            d_.wait()

    def process(c, b, fire_gnext, fire_inext):
        if fire_gnext:
            drain(idx_descs(c + 1, 1 - b))
            fire(gather_descs(c + 1, 1 - b))
        drain(gather_descs(c, b))

        def mul(i, c2):
            w16 = wv[b, pl.ds(i * 16, 16)]
            for l in range(16):
                r = i * 16 + l
                u = rows[b, r, pl.ds(0, 16)]
                fa = plsc.bitcast(lax.shift_left(u, 16), jnp.float32)
                fb = plsc.bitcast(u & jnp.int32(-65536), jnp.float32)
                frows[r, pl.ds(0, 16)] = fa * w16[l]
                frows[r, pl.ds(16, 16)] = fb * w16[l]
            return c2
        lax.fori_loop(0, S2_CH // 16, mul, 0)
        for j in range(_NJ):
            pltpu.sync_copy(frows.at[pl.ds(j * 128, 128)],
                            acc.at[dstv.at[_NJ * b + j]], add=True)
        if fire_inext:
            fire(idx_descs(c + 2, b))

    fire(idx_descs(0, 0))
    drain(idx_descs(0, 0))
    fire(gather_descs(0, 0))
    fire(idx_descs(1, 1))

    def body(cj, carry):
        c = cj * 2
        process(c, 0, True, True)
        process(c + 1, 1, True, True)
        return carry
    lax.fori_loop(0, _NCH // 2 - 1, body, 0)
    process(_NCH - 2, 0, True, False)
    process(_NCH - 1, 1, False, False)
    plsc.subcore_barrier()

    for k in range(NSTR // SZB):
        pltpu.sync_copy(acc.at[pl.ds(sub * NSTR + k * SZB, SZB)],
                        frows.at[pl.ds(0, SZB)])
        pltpu.sync_copy(frows.at[pl.ds(0, SZB)],
                        out_ref.at[core, pl.ds(sub * NSTR + k * SZB, SZB)])


@functools.cache
def _build_s2():
    return pl.kernel(
        _s2_body,
        out_type=jax.ShapeDtypeStruct((NC, NPAD, HID), jnp.float32),
        mesh=_mesh(),
        compiler_params=pltpu.CompilerParams(use_tc_tiling_on_sc=False,
                                             needs_layout_passes=False),
        scratch_types=[
            pltpu.VMEM((2 * _NJ, 128), jnp.int32),
            pltpu.VMEM((2 * _NJ, 128), jnp.int32),
            pltpu.VMEM((2, S2_CH), jnp.float32),
            pltpu.VMEM((2, S2_CH, HID // 2), jnp.int32),
            pltpu.VMEM((S2_CH, HID), jnp.float32),
            pltpu.SemaphoreType.DMA,
            pltpu.SemaphoreType.DMA,
            pltpu.SemaphoreType.DMA,
            pltpu.SemaphoreType.DMA,
            pltpu.VMEM_SHARED((NPAD, HID), jnp.float32),
        ])


def _s2(*args):
    return _build_s2()(*args)


_PPT = PPAD // NW


def _s3_body(x2_ref, batch_ref, out_ref, batchv, rows, acc):
    core = lax.axis_index("c")
    sub = lax.axis_index("s")
    wid = sub * NC + core

    def zrow(i, carry):
        rows[i, pl.ds(0, 16)] = jnp.zeros((16,), jnp.float32)
        rows[i, pl.ds(16, 16)] = jnp.zeros((16,), jnp.float32)
        return carry
    lax.fori_loop(0, GSTR, zrow, 0)
    pltpu.sync_copy(rows.at[pl.ds(0, GSTR)], acc.at[pl.ds(sub * GSTR, GSTR)])
    plsc.subcore_barrier()

    pltpu.sync_copy(x2_ref.at[pl.ds(wid * _PPT, _PPT)], rows)
    for j in range(_PPT // 128):
        pltpu.sync_copy(batch_ref.at[pl.ds(wid * _PPT + j * 128, 128)],
                        batchv.at[j])
    for j in range(_PPT // 128):
        pltpu.sync_copy(rows.at[pl.ds(j * 128, 128)],
                        acc.at[batchv.at[j]], add=True)
    plsc.subcore_barrier()

    pltpu.sync_copy(acc.at[pl.ds(sub * GSTR, GSTR)], rows.at[pl.ds(0, GSTR)])
    pltpu.sync_copy(rows.at[pl.ds(0, GSTR)],
                    out_ref.at[core, pl.ds(sub * GSTR, GSTR)])


@functools.cache
def _build_s3():
    return pl.kernel(
        _s3_body,
        out_type=jax.ShapeDtypeStruct((NC, GPAD, OUT), jnp.float32),
        mesh=_mesh(),
        compiler_params=pltpu.CompilerParams(use_tc_tiling_on_sc=False),
        scratch_types=[
            pltpu.VMEM((_PPT // 128, 128), jnp.int32),
            pltpu.VMEM((_PPT, OUT), jnp.float32),
            pltpu.VMEM_SHARED((GPAD, OUT), jnp.float32),
        ])


def _s3(*args):
    return _build_s3()(*args)


def kernel(x, edge_index, edge_attr, batch, Wih_s, Whh_s, bih_s, bhh_s,
           Watt_s, Wfc_s, bfc_s, Wih_n, Whh_n, bih_n, bhh_n, Watt_n, Wfc_n,
           bfc_n, Wg1, bg1, Wg2, bg2, Wls, bls):
    f32 = jnp.float32
    x = x.astype(f32)

    xT_node = jnp.concatenate(
        [x.T, jnp.zeros((LAG, NPAD - N), f32)], 1).reshape(LAG, 1, NPAD)
    xsT = x.reshape(NG, 3, LAG).transpose(2, 1, 0)
    xsT = jnp.concatenate([xsT, jnp.zeros((LAG, 3, GPAD - NG), f32)], 2)
    src = edge_index[0].astype(jnp.int32)
    dst = edge_index[1].astype(jnp.int32)
    epad = E_PAD - E
    src_p = jnp.concatenate([src, jnp.zeros((epad,), jnp.int32)])
    dst_p = jnp.concatenate([dst, jnp.zeros((epad,), jnp.int32)])
    dst2 = dst_p.reshape(-1, 128)
    w_p = jnp.concatenate([edge_attr.astype(f32), jnp.zeros((epad,), f32)])
    batch1 = jnp.concatenate(
        [batch.astype(jnp.int32), jnp.full((PPAD - N,), NG, jnp.int32)])

    b_n = (bih_n + bhh_n).reshape(4 * HID, 1)
    b_s = (bih_s + bhh_s).reshape(4 * HID, 1)

    emb = _attlstm(xT_node, Wih_n, Whh_n, b_n,
                   Watt_n, Wfc_n, bfc_n.reshape(OUT, 1))

    src_e, dst_e, w_e = _s0(src_p, dst_p, w_p)
    degp, cntp = _s1(dst2, w_p.reshape(-1, 128), batch1)
    degp = degp.reshape(NC, NPAD)
    cntp = cntp.reshape(NC, GPAD)

    dis, h1s, t1 = _g0(degp.T, emb, Wg1)
    agg1p = _s2(t1, src_e, dst_e, w_e)

    xsT_b, _ = lax.optimization_barrier((xsT, h1s))
    x1s = _attlstm(xsT_b, Wih_s, Whh_s, b_s,
                   Watt_s, Wfc_s, bfc_s.reshape(OUT, 1))

    h2s, t2 = _g1(agg1p, dis, h1s, bg1.reshape(1, HID), Wg2)

    agg2p = _s2(t2, src_e, dst_e, w_e)
    x2 = _g2(agg2p, dis, h2s, bg2.reshape(1, OUT))

    x2_pool = jnp.concatenate([x2, jnp.zeros((PPAD - NPAD, OUT), f32)], 0)
    ssump = _s3(x2_pool, batch1)
    pred = _g3(ssump, cntp.T, x1s, Wls[:OUT].astype(f32),
               Wls[OUT:].astype(f32), bls.reshape(1, 1))
    return pred[:NG, 0]


# speedup vs baseline: 25.3135x; 1.0301x over previous
import functools

import jax
import jax.numpy as jnp
from jax import lax
from jax.experimental import pallas as pl
from jax.experimental.pallas import tpu as pltpu
from jax.experimental.pallas import tpu_sc as plsc

N = 50001
E = 1600032
NG = 16667
LAG = 30
HID = 32
OUT = 32

NC = 2
NS = 16
NW = NC * NS

NPAD = 50176
GPAD = 16896
PPAD = 53248
E_PAD = 1638400
EPT = E_PAD // NW

R_LSTM = 512
S1_CH = 1024
S2_CH = 512
SZB = 224
NSTR = NPAD // NS
GSTR = GPAD // NS

@functools.cache
def _mesh():
    return plsc.VectorSubcoreMesh(
        core_axis_name="c", subcore_axis_name="s",
        num_cores=NC, num_subcores=NS)


def _attlstm_body(x_ref, wih_ref, whh_ref, b_ref, watt_ref, wfc_ref,
                  bfc_ref, out_ref, hs_ref):
    R = x_ref.shape[2]
    D = x_ref.shape[1]
    whh = whh_ref[...]
    b = b_ref[...]
    hT = jnp.zeros((HID, R), jnp.float32)
    cT = jnp.zeros((HID, R), jnp.float32)
    scores = []
    for t in range(LAG):
        xtT = x_ref[t]
        if D == 1:
            gx = wih_ref[...] * xtT
        else:
            gx = jnp.dot(wih_ref[...], xtT,
                         preferred_element_type=jnp.float32)
        g = gx + jnp.dot(whh, hT, preferred_element_type=jnp.float32) + b
        i = jax.nn.sigmoid(g[0:HID])
        f = jax.nn.sigmoid(g[HID:2 * HID])
        gg = jnp.tanh(g[2 * HID:3 * HID])
        o = jax.nn.sigmoid(g[3 * HID:4 * HID])
        cT = f * cT + i * gg
        hT = o * jnp.tanh(cT)
        hs_ref[t] = hT
        scores.append(jnp.dot(watt_ref[t:t + 1, :], hT,
                              preferred_element_type=jnp.float32))
    s = jnp.concatenate(scores, axis=0)
    m = jnp.max(s, axis=0, keepdims=True)
    e = jnp.exp(s - m)
    a = e / jnp.sum(e, axis=0, keepdims=True)
    attT = jnp.zeros((HID, R), jnp.float32)
    for t in range(LAG):
        attT = attT + a[t:t + 1, :] * hs_ref[t]
    outT = (jnp.dot(wfc_ref[...], attT, preferred_element_type=jnp.float32)
            + bfc_ref[...])
    out_ref[...] = jnp.maximum(outT, 0.0)


def _attlstm(xpT, wih, whh, b2, watt, wfc, bfc2):
    _, D, B = xpT.shape
    grid = B // R_LSTM
    outT = pl.pallas_call(
        _attlstm_body,
        grid=(grid,),
        in_specs=[
            pl.BlockSpec((LAG, D, R_LSTM), lambda i: (0, 0, i)),
            pl.BlockSpec((4 * HID, D), lambda i: (0, 0)),
            pl.BlockSpec((4 * HID, HID), lambda i: (0, 0)),
            pl.BlockSpec((4 * HID, 1), lambda i: (0, 0)),
            pl.BlockSpec((LAG, HID), lambda i: (0, 0)),
            pl.BlockSpec((OUT, HID), lambda i: (0, 0)),
            pl.BlockSpec((OUT, 1), lambda i: (0, 0)),
        ],
        out_specs=pl.BlockSpec((OUT, R_LSTM), lambda i: (0, i)),
        out_shape=jax.ShapeDtypeStruct((OUT, B), jnp.float32),
        scratch_shapes=[pltpu.VMEM((LAG, HID, R_LSTM), jnp.float32)],
    )(xpT, wih, whh, b2, watt, wfc, bfc2)
    return outT.T


_RG = 3584


def _pack_bf16(hs):
    a = jax.lax.convert_element_type(hs[:, 0:HID // 2], jnp.bfloat16)
    b = jax.lax.convert_element_type(hs[:, HID // 2:], jnp.bfloat16)
    au = jax.lax.bitcast_convert_type(a, jnp.uint16).astype(jnp.uint32)
    bu = jax.lax.bitcast_convert_type(b, jnp.uint16).astype(jnp.uint32)
    return jax.lax.bitcast_convert_type(au | (bu << 16), jnp.int32)


def _g0_body(degp_ref, emb_ref, wg1_ref, dis_ref, h1s_ref, t1_ref):
    deg = degp_ref[:, 0:1] + degp_ref[:, 1:2] + 1.0
    dis = lax.rsqrt(deg)
    h1 = jnp.dot(emb_ref[...], wg1_ref[...], preferred_element_type=jnp.float32)
    dis_ref[...] = dis
    h1s = dis * h1
    h1s_ref[...] = h1s
    t1_ref[...] = _pack_bf16(h1s)


def _g0(degpT, emb, wg1):
    grid = NPAD // _RG
    return pl.pallas_call(
        _g0_body,
        grid=(grid,),
        in_specs=[
            pl.BlockSpec((_RG, NC), lambda i: (i, 0)),
            pl.BlockSpec((_RG, HID), lambda i: (i, 0)),
            pl.BlockSpec((HID, HID), lambda i: (0, 0)),
        ],
        out_specs=[
            pl.BlockSpec((_RG, 1), lambda i: (i, 0)),
            pl.BlockSpec((_RG, HID), lambda i: (i, 0)),
            pl.BlockSpec((_RG, HID // 2), lambda i: (i, 0)),
        ],
        out_shape=[
            jax.ShapeDtypeStruct((NPAD, 1), jnp.float32),
            jax.ShapeDtypeStruct((NPAD, HID), jnp.float32),
            jax.ShapeDtypeStruct((NPAD, HID // 2), jnp.int32),
        ],
    )(degpT, emb, wg1)


def _g1_body(aggp_ref, dis_ref, h1s_ref, bg1_ref, wg2_ref, h2s_ref, t2_ref):
    dis = dis_ref[...]
    x2a = dis * (aggp_ref[0] + aggp_ref[1] + h1s_ref[...]) + bg1_ref[...]
    h2 = jnp.dot(x2a, wg2_ref[...], preferred_element_type=jnp.float32)
    h2s = dis * h2
    h2s_ref[...] = h2s
    t2_ref[...] = _pack_bf16(h2s)


def _g1(aggp, dis, h1s, bg1, wg2):
    grid = NPAD // _RG
    return pl.pallas_call(
        _g1_body,
        grid=(grid,),
        in_specs=[
            pl.BlockSpec((NC, _RG, HID), lambda i: (0, i, 0)),
            pl.BlockSpec((_RG, 1), lambda i: (i, 0)),
            pl.BlockSpec((_RG, HID), lambda i: (i, 0)),
            pl.BlockSpec((1, HID), lambda i: (0, 0)),
            pl.BlockSpec((HID, OUT), lambda i: (0, 0)),
        ],
        out_specs=[
            pl.BlockSpec((_RG, OUT), lambda i: (i, 0)),
            pl.BlockSpec((_RG, HID // 2), lambda i: (i, 0)),
        ],
        out_shape=[
            jax.ShapeDtypeStruct((NPAD, OUT), jnp.float32),
            jax.ShapeDtypeStruct((NPAD, HID // 2), jnp.int32),
        ],
    )(aggp, dis, h1s, bg1, wg2)


def _g2_body(aggp_ref, dis_ref, h2s_ref, bg2_ref, x2_ref):
    dis = dis_ref[...]
    x2_ref[...] = dis * (aggp_ref[0] + aggp_ref[1] + h2s_ref[...]) + bg2_ref[...]


def _g2(aggp, dis, h2s, bg2):
    grid = NPAD // _RG
    return pl.pallas_call(
        _g2_body,
        grid=(grid,),
        in_specs=[
            pl.BlockSpec((NC, _RG, OUT), lambda i: (0, i, 0)),
            pl.BlockSpec((_RG, 1), lambda i: (i, 0)),
            pl.BlockSpec((_RG, OUT), lambda i: (i, 0)),
            pl.BlockSpec((1, OUT), lambda i: (0, 0)),
        ],
        out_specs=pl.BlockSpec((_RG, OUT), lambda i: (i, 0)),
        out_shape=jax.ShapeDtypeStruct((NPAD, OUT), jnp.float32),
    )(aggp, dis, h2s, bg2)


_RG3 = 2112


def _g3_body(ssump_ref, cntp_ref, x1s_ref, wa_ref, wb_ref, bls_ref, out_ref):
    cnt = jnp.maximum(cntp_ref[:, 0:1] + cntp_ref[:, 1:2], 1.0)
    x2n = (ssump_ref[0] + ssump_ref[1]) / cnt
    pred = (jnp.dot(x1s_ref[...], wa_ref[...], preferred_element_type=jnp.float32)
            + jnp.dot(x2n, wb_ref[...], preferred_element_type=jnp.float32)
            + bls_ref[...])
    out_ref[...] = jnp.maximum(pred, 0.0)


def _g3(ssump, cntpT, x1s, wa, wb, bls2):
    grid = GPAD // _RG3
    return pl.pallas_call(
        _g3_body,
        grid=(grid,),
        in_specs=[
            pl.BlockSpec((NC, _RG3, OUT), lambda i: (0, i, 0)),
            pl.BlockSpec((_RG3, NC), lambda i: (i, 0)),
            pl.BlockSpec((_RG3, OUT), lambda i: (i, 0)),
            pl.BlockSpec((OUT, 1), lambda i: (0, 0)),
            pl.BlockSpec((OUT, 1), lambda i: (0, 0)),
            pl.BlockSpec((1, 1), lambda i: (0, 0)),
        ],
        out_specs=pl.BlockSpec((_RG3, 1), lambda i: (i, 0)),
        out_shape=jax.ShapeDtypeStruct((GPAD, 1), jnp.float32),
    )(ssump, cntpT, x1s, wa, wb, bls2)


_PCH = 6400


def _s0_body(src_ref, dst_ref, w_ref, srco_ref, dsto_ref, wo_ref, bi, bf):
    core = lax.axis_index("c")
    sub = lax.axis_index("s")
    wid = sub * NC + core

    def chunk(ci, carry):
        base = wid * EPT + ci * _PCH
        for a, o, b in ((src_ref, srco_ref, bi), (dst_ref, dsto_ref, bi),
                        (w_ref, wo_ref, bf)):
            pltpu.sync_copy(a.at[pl.ds(base, _PCH)], b)
            pltpu.sync_copy(b, o.at[pl.ds(base, _PCH)])
        return carry
    lax.fori_loop(0, EPT // _PCH, chunk, 0)


@functools.cache
def _build_s0():
    return pl.kernel(
        _s0_body,
        out_type=(jax.ShapeDtypeStruct((E_PAD,), jnp.int32),
                  jax.ShapeDtypeStruct((E_PAD,), jnp.int32),
                  jax.ShapeDtypeStruct((E_PAD,), jnp.float32)),
        mesh=_mesh(),
        compiler_params=pltpu.CompilerParams(use_tc_tiling_on_sc=False),
        scratch_types=[pltpu.VMEM((_PCH,), jnp.int32),
                       pltpu.VMEM((_PCH,), jnp.float32)])


def _s0(*args):
    return _build_s0()(*args)


def _memset(ref, n, val):
    def step(i, carry):
        ref[pl.ds(i * 16, 16)] = jnp.full((16,), val, jnp.float32)
        return carry
    lax.fori_loop(0, n // 16, step, 0)


def _s1_body(dst_ref, w_ref, batch_ref, degout_ref, cntout_ref,
             dstv, wv, batchv, zbuf, onesv, dacc, cacc):
    core = lax.axis_index("c")
    sub = lax.axis_index("s")
    wid = sub * NC + core
    _memset(zbuf, NSTR, 0.0)
    _memset(onesv, 128, 1.0)
    pltpu.sync_copy(zbuf.at[pl.ds(0, NSTR)], dacc.at[pl.ds(sub * NSTR, NSTR)])
    pltpu.sync_copy(zbuf.at[pl.ds(0, GSTR)], cacc.at[pl.ds(sub * GSTR, GSTR)])
    plsc.subcore_barrier()

    def echunk(ci, carry):
        row0 = wid * (EPT // 128) + ci * (S1_CH // 128)
        pltpu.sync_copy(dst_ref.at[pl.ds(row0, S1_CH // 128)], dstv)
        pltpu.sync_copy(w_ref.at[pl.ds(row0, S1_CH // 128)], wv)
        for j in range(S1_CH // 128):
            pltpu.sync_copy(wv.at[j], dacc.at[dstv.at[j]], add=True)
        return carry
    lax.fori_loop(0, EPT // S1_CH, echunk, 0)

    for j in range(PPAD // NW // 128):
        pltpu.sync_copy(batch_ref.at[pl.ds(wid * (PPAD // NW) + j * 128, 128)],
                        batchv.at[j])
    for j in range(PPAD // NW // 128):
        pltpu.sync_copy(onesv, cacc.at[batchv.at[j]], add=True)
    plsc.subcore_barrier()

    pltpu.sync_copy(dacc.at[pl.ds(sub * NSTR, NSTR)], zbuf.at[pl.ds(0, NSTR)])
    pltpu.sync_copy(zbuf.at[pl.ds(0, NSTR)],
                    degout_ref.at[pl.ds(core * NPAD + sub * NSTR, NSTR)])
    pltpu.sync_copy(cacc.at[pl.ds(sub * GSTR, GSTR)], zbuf.at[pl.ds(0, GSTR)])
    pltpu.sync_copy(zbuf.at[pl.ds(0, GSTR)],
                    cntout_ref.at[pl.ds(core * GPAD + sub * GSTR, GSTR)])


@functools.cache
def _build_s1():
    return pl.kernel(
        _s1_body,
        out_type=(jax.ShapeDtypeStruct((NC * NPAD,), jnp.float32),
                  jax.ShapeDtypeStruct((NC * GPAD,), jnp.float32)),
        mesh=_mesh(),
        compiler_params=pltpu.CompilerParams(use_tc_tiling_on_sc=False),
        scratch_types=[
            pltpu.VMEM((S1_CH // 128, 128), jnp.int32),
            pltpu.VMEM((S1_CH // 128, 128), jnp.float32),
            pltpu.VMEM((PPAD // NW // 128, 128), jnp.int32),
            pltpu.VMEM((NSTR,), jnp.float32),
            pltpu.VMEM((128,), jnp.float32),
            pltpu.VMEM_SHARED((NPAD,), jnp.float32),
            pltpu.VMEM_SHARED((GPAD,), jnp.float32),
        ])


def _s1(*args):
    return _build_s1()(*args)


_NJ = S2_CH // 128
_NCH = EPT // S2_CH


def _s2_body(table_ref, src_ref, dst_ref, w_ref, out_ref,
             srcv, dstv, wv, rows, frows, semr0, semr1, semi0, semi1, acc):
    core = lax.axis_index("c")
    sub = lax.axis_index("s")
    wid = sub * NC + core
    semr = (semr0, semr1)
    semi = (semi0, semi1)

    def zrow(i, carry):
        frows[i, pl.ds(0, 16)] = jnp.zeros((16,), jnp.float32)
        frows[i, pl.ds(16, 16)] = jnp.zeros((16,), jnp.float32)
        return carry
    lax.fori_loop(0, SZB, zrow, 0)
    for k in range(NSTR // SZB):
        pltpu.sync_copy(frows.at[pl.ds(0, SZB)],
                        acc.at[pl.ds(sub * NSTR + k * SZB, SZB)])
    plsc.subcore_barrier()

    def idx_descs(c, b):
        base = wid * EPT + c * S2_CH
        ds_ = []
        for j in range(_NJ):
            ds_.append(pltpu.make_async_copy(
                src_ref.at[pl.ds(base + j * 128, 128)],
                srcv.at[_NJ * b + j], semi[b]))
            ds_.append(pltpu.make_async_copy(
                dst_ref.at[pl.ds(base + j * 128, 128)],
                dstv.at[_NJ * b + j], semi[b]))
        ds_.append(pltpu.make_async_copy(
            w_ref.at[pl.ds(base, S2_CH)], wv.at[b], semi[b]))
        return ds_

    def gather_descs(c, b):
        return [pltpu.make_async_copy(table_ref.at[srcv.at[_NJ * b + j]],
                                      rows.at[b, pl.ds(j * 128, 128)],
                                      semr[b])
                for j in range(_NJ)]

    def fire(descs):
        for d_ in descs:
            d_.start()

    def drain(descs):
        for d_ in descs:
            d_.wait()

    def process(c, b, fire_gnext, fire_inext):
        if fire_gnext:
            drain(idx_descs(c + 1, 1 - b))
            fire(gather_descs(c + 1, 1 - b))
        drain(gather_descs(c, b))
        for j in range(_NJ):
            def mul(i, c2, j=j):
                w16 = wv[b, pl.ds(j * 128 + i * 16, 16)]
                for l in range(16):
                    r = j * 128 + i * 16 + l
                    u = rows[b, r, pl.ds(0, 16)]
                    fa = plsc.bitcast(lax.shift_left(u, 16), jnp.float32)
                    fb = plsc.bitcast(u & jnp.int32(-65536), jnp.float32)
                    frows[i * 16 + l, pl.ds(0, 16)] = fa * w16[l]
                    frows[i * 16 + l, pl.ds(16, 16)] = fb * w16[l]
                return c2
            lax.fori_loop(0, 8, mul, 0)
            pltpu.sync_copy(frows.at[pl.ds(0, 128)],
                            acc.at[dstv.at[_NJ * b + j]], add=True)
        if fire_inext:
            fire(idx_descs(c + 2, b))

    fire(idx_descs(0, 0))
    drain(idx_descs(0, 0))
    fire(gather_descs(0, 0))
    fire(idx_descs(1, 1))

    def body(cj, carry):
        c = cj * 2
        process(c, 0, True, True)
        process(c + 1, 1, True, True)
        return carry
    lax.fori_loop(0, _NCH // 2 - 1, body, 0)
    process(_NCH - 2, 0, True, False)
    process(_NCH - 1, 1, False, False)
    plsc.subcore_barrier()

    for k in range(NSTR // SZB):
        pltpu.sync_copy(acc.at[pl.ds(sub * NSTR + k * SZB, SZB)],
                        frows.at[pl.ds(0, SZB)])
        pltpu.sync_copy(frows.at[pl.ds(0, SZB)],
                        out_ref.at[core, pl.ds(sub * NSTR + k * SZB, SZB)])


@functools.cache
def _build_s2():
    return pl.kernel(
        _s2_body,
        out_type=jax.ShapeDtypeStruct((NC, NPAD, HID), jnp.float32),
        mesh=_mesh(),
        compiler_params=pltpu.CompilerParams(use_tc_tiling_on_sc=False,
                                             needs_layout_passes=False),
        scratch_types=[
            pltpu.VMEM((2 * _NJ, 128), jnp.int32),
            pltpu.VMEM((2 * _NJ, 128), jnp.int32),
            pltpu.VMEM((2, S2_CH), jnp.float32),
            pltpu.VMEM((2, S2_CH, HID // 2), jnp.int32),
            pltpu.VMEM((SZB, HID), jnp.float32),
            pltpu.SemaphoreType.DMA,
            pltpu.SemaphoreType.DMA,
            pltpu.SemaphoreType.DMA,
            pltpu.SemaphoreType.DMA,
            pltpu.VMEM_SHARED((NPAD, HID), jnp.float32),
        ])


def _s2(*args):
    return _build_s2()(*args)


_PPT = PPAD // NW


def _s3_body(x2_ref, batch_ref, out_ref, batchv, rows, acc):
    core = lax.axis_index("c")
    sub = lax.axis_index("s")
    wid = sub * NC + core

    def zrow(i, carry):
        rows[i, pl.ds(0, 16)] = jnp.zeros((16,), jnp.float32)
        rows[i, pl.ds(16, 16)] = jnp.zeros((16,), jnp.float32)
        return carry
    lax.fori_loop(0, GSTR, zrow, 0)
    pltpu.sync_copy(rows.at[pl.ds(0, GSTR)], acc.at[pl.ds(sub * GSTR, GSTR)])
    plsc.subcore_barrier()

    pltpu.sync_copy(x2_ref.at[pl.ds(wid * _PPT, _PPT)], rows)
    for j in range(_PPT // 128):
        pltpu.sync_copy(batch_ref.at[pl.ds(wid * _PPT + j * 128, 128)],
                        batchv.at[j])
    for j in range(_PPT // 128):
        pltpu.sync_copy(rows.at[pl.ds(j * 128, 128)],
                        acc.at[batchv.at[j]], add=True)
    plsc.subcore_barrier()

    pltpu.sync_copy(acc.at[pl.ds(sub * GSTR, GSTR)], rows.at[pl.ds(0, GSTR)])
    pltpu.sync_copy(rows.at[pl.ds(0, GSTR)],
                    out_ref.at[core, pl.ds(sub * GSTR, GSTR)])


@functools.cache
def _build_s3():
    return pl.kernel(
        _s3_body,
        out_type=jax.ShapeDtypeStruct((NC, GPAD, OUT), jnp.float32),
        mesh=_mesh(),
        compiler_params=pltpu.CompilerParams(use_tc_tiling_on_sc=False),
        scratch_types=[
            pltpu.VMEM((_PPT // 128, 128), jnp.int32),
            pltpu.VMEM((_PPT, OUT), jnp.float32),
            pltpu.VMEM_SHARED((GPAD, OUT), jnp.float32),
        ])


def _s3(*args):
    return _build_s3()(*args)


def kernel(x, edge_index, edge_attr, batch, Wih_s, Whh_s, bih_s, bhh_s,
           Watt_s, Wfc_s, bfc_s, Wih_n, Whh_n, bih_n, bhh_n, Watt_n, Wfc_n,
           bfc_n, Wg1, bg1, Wg2, bg2, Wls, bls):
    f32 = jnp.float32
    x = x.astype(f32)

    xT_node = jnp.concatenate(
        [x.T, jnp.zeros((LAG, NPAD - N), f32)], 1).reshape(LAG, 1, NPAD)
    xsT = x.reshape(NG, 3, LAG).transpose(2, 1, 0)
    xsT = jnp.concatenate([xsT, jnp.zeros((LAG, 3, GPAD - NG), f32)], 2)
    src = edge_index[0].astype(jnp.int32)
    dst = edge_index[1].astype(jnp.int32)
    epad = E_PAD - E
    src_p = jnp.concatenate([src, jnp.zeros((epad,), jnp.int32)])
    dst_p = jnp.concatenate([dst, jnp.zeros((epad,), jnp.int32)])
    dst2 = dst_p.reshape(-1, 128)
    w_p = jnp.concatenate([edge_attr.astype(f32), jnp.zeros((epad,), f32)])
    batch1 = jnp.concatenate(
        [batch.astype(jnp.int32), jnp.full((PPAD - N,), NG, jnp.int32)])

    b_n = (bih_n + bhh_n).reshape(4 * HID, 1)
    b_s = (bih_s + bhh_s).reshape(4 * HID, 1)

    emb = _attlstm(xT_node, Wih_n, Whh_n, b_n,
                   Watt_n, Wfc_n, bfc_n.reshape(OUT, 1))

    src_e, dst_e, w_e = _s0(src_p, dst_p, w_p)
    degp, cntp = _s1(dst2, w_p.reshape(-1, 128), batch1)
    degp = degp.reshape(NC, NPAD)
    cntp = cntp.reshape(NC, GPAD)

    dis, h1s, t1 = _g0(degp.T, emb, Wg1)
    agg1p = _s2(t1, src_e, dst_e, w_e)

    xsT_b, _ = lax.optimization_barrier((xsT, h1s))
    x1s = _attlstm(xsT_b, Wih_s, Whh_s, b_s,
                   Watt_s, Wfc_s, bfc_s.reshape(OUT, 1))

    h2s, t2 = _g1(agg1p, dis, h1s, bg1.reshape(1, HID), Wg2)

    agg2p = _s2(t2, src_e, dst_e, w_e)
    x2 = _g2(agg2p, dis, h2s, bg2.reshape(1, OUT))

    x2_pool = jnp.concatenate([x2, jnp.zeros((PPAD - NPAD, OUT), f32)], 0)
    ssump = _s3(x2_pool, batch1)
    pred = _g3(ssump, cntp.T, x1s, Wls[:OUT].astype(f32),
               Wls[OUT:].astype(f32), bls.reshape(1, 1))
    return pred[:NG, 0]


# speedup vs baseline: 27.8030x; 1.0983x over previous
import functools

import jax
import jax.numpy as jnp
from jax import lax
from jax.experimental import pallas as pl
from jax.experimental.pallas import tpu as pltpu
from jax.experimental.pallas import tpu_sc as plsc

N = 50001
E = 1600032
NG = 16667
LAG = 30
HID = 32
OUT = 32

NC = 2
NS = 16
NW = NC * NS

NPAD = 50176
GPAD = 16896
PPAD = 53248
E_PAD = 1638400
EPT = E_PAD // NW

R_LSTM = 512
S1_CH = 1024
S2_CH = 512
SZB = 224
NSTR = NPAD // NS
GSTR = GPAD // NS

@functools.cache
def _mesh():
    return plsc.VectorSubcoreMesh(
        core_axis_name="c", subcore_axis_name="s",
        num_cores=NC, num_subcores=NS)


def _attlstm_body(x_ref, wih_ref, whh_ref, b_ref, watt_ref, wfc_ref,
                  bfc_ref, out_ref, hs_ref):
    R = x_ref.shape[2]
    D = x_ref.shape[1]
    whh = whh_ref[...]
    b = b_ref[...]
    hT = jnp.zeros((HID, R), jnp.float32)
    cT = jnp.zeros((HID, R), jnp.float32)
    scores = []
    for t in range(LAG):
        xtT = x_ref[t]
        if D == 1:
            gx = wih_ref[...] * xtT
        else:
            gx = jnp.dot(wih_ref[...], xtT,
                         preferred_element_type=jnp.float32)
        g = gx + jnp.dot(whh, hT, preferred_element_type=jnp.float32) + b
        i = jax.nn.sigmoid(g[0:HID])
        f = jax.nn.sigmoid(g[HID:2 * HID])
        gg = jnp.tanh(g[2 * HID:3 * HID])
        o = jax.nn.sigmoid(g[3 * HID:4 * HID])
        cT = f * cT + i * gg
        hT = o * jnp.tanh(cT)
        hs_ref[t] = hT
        scores.append(jnp.dot(watt_ref[t:t + 1, :], hT,
                              preferred_element_type=jnp.float32))
    s = jnp.concatenate(scores, axis=0)
    m = jnp.max(s, axis=0, keepdims=True)
    e = jnp.exp(s - m)
    a = e / jnp.sum(e, axis=0, keepdims=True)
    attT = jnp.zeros((HID, R), jnp.float32)
    for t in range(LAG):
        attT = attT + a[t:t + 1, :] * hs_ref[t]
    outT = (jnp.dot(wfc_ref[...], attT, preferred_element_type=jnp.float32)
            + bfc_ref[...])
    out_ref[...] = jnp.maximum(outT, 0.0)


def _attlstm(xpT, wih, whh, b2, watt, wfc, bfc2, R_LSTM=512):
    _, D, B = xpT.shape
    grid = B // R_LSTM
    outT = pl.pallas_call(
        _attlstm_body,
        grid=(grid,),
        in_specs=[
            pl.BlockSpec((LAG, D, R_LSTM), lambda i: (0, 0, i)),
            pl.BlockSpec((4 * HID, D), lambda i: (0, 0)),
            pl.BlockSpec((4 * HID, HID), lambda i: (0, 0)),
            pl.BlockSpec((4 * HID, 1), lambda i: (0, 0)),
            pl.BlockSpec((LAG, HID), lambda i: (0, 0)),
            pl.BlockSpec((OUT, HID), lambda i: (0, 0)),
            pl.BlockSpec((OUT, 1), lambda i: (0, 0)),
        ],
        out_specs=pl.BlockSpec((OUT, R_LSTM), lambda i: (0, i)),
        out_shape=jax.ShapeDtypeStruct((OUT, B), jnp.float32),
        scratch_shapes=[pltpu.VMEM((LAG, HID, R_LSTM), jnp.float32)],
    )(xpT, wih, whh, b2, watt, wfc, bfc2)
    return outT.T


_RG = 3584


def _pack_bf16(hs):
    a = jax.lax.convert_element_type(hs[:, 0:HID // 2], jnp.bfloat16)
    b = jax.lax.convert_element_type(hs[:, HID // 2:], jnp.bfloat16)
    au = jax.lax.bitcast_convert_type(a, jnp.uint16).astype(jnp.uint32)
    bu = jax.lax.bitcast_convert_type(b, jnp.uint16).astype(jnp.uint32)
    return jax.lax.bitcast_convert_type(au | (bu << 16), jnp.int32)


def _g0_body(degp_ref, emb_ref, wg1_ref, dis_ref, h1s_ref, t1_ref):
    deg = degp_ref[:, 0:1] + degp_ref[:, 1:2] + 1.0
    dis = lax.rsqrt(deg)
    h1 = jnp.dot(emb_ref[...], wg1_ref[...], preferred_element_type=jnp.float32)
    dis_ref[...] = dis
    h1s = dis * h1
    h1s_ref[...] = h1s
    t1_ref[...] = _pack_bf16(h1s)


def _g0(degpT, emb, wg1):
    grid = NPAD // _RG
    return pl.pallas_call(
        _g0_body,
        grid=(grid,),
        in_specs=[
            pl.BlockSpec((_RG, NC), lambda i: (i, 0)),
            pl.BlockSpec((_RG, HID), lambda i: (i, 0)),
            pl.BlockSpec((HID, HID), lambda i: (0, 0)),
        ],
        out_specs=[
            pl.BlockSpec((_RG, 1), lambda i: (i, 0)),
            pl.BlockSpec((_RG, HID), lambda i: (i, 0)),
            pl.BlockSpec((_RG, HID // 2), lambda i: (i, 0)),
        ],
        out_shape=[
            jax.ShapeDtypeStruct((NPAD, 1), jnp.float32),
            jax.ShapeDtypeStruct((NPAD, HID), jnp.float32),
            jax.ShapeDtypeStruct((NPAD, HID // 2), jnp.int32),
        ],
    )(degpT, emb, wg1)


def _g1_body(aggp_ref, dis_ref, h1s_ref, bg1_ref, wg2_ref, h2s_ref, t2_ref):
    dis = dis_ref[...]
    x2a = dis * (aggp_ref[0] + aggp_ref[1] + h1s_ref[...]) + bg1_ref[...]
    h2 = jnp.dot(x2a, wg2_ref[...], preferred_element_type=jnp.float32)
    h2s = dis * h2
    h2s_ref[...] = h2s
    t2_ref[...] = _pack_bf16(h2s)


def _g1(aggp, dis, h1s, bg1, wg2):
    grid = NPAD // _RG
    return pl.pallas_call(
        _g1_body,
        grid=(grid,),
        in_specs=[
            pl.BlockSpec((NC, _RG, HID), lambda i: (0, i, 0)),
            pl.BlockSpec((_RG, 1), lambda i: (i, 0)),
            pl.BlockSpec((_RG, HID), lambda i: (i, 0)),
            pl.BlockSpec((1, HID), lambda i: (0, 0)),
            pl.BlockSpec((HID, OUT), lambda i: (0, 0)),
        ],
        out_specs=[
            pl.BlockSpec((_RG, OUT), lambda i: (i, 0)),
            pl.BlockSpec((_RG, HID // 2), lambda i: (i, 0)),
        ],
        out_shape=[
            jax.ShapeDtypeStruct((NPAD, OUT), jnp.float32),
            jax.ShapeDtypeStruct((NPAD, HID // 2), jnp.int32),
        ],
    )(aggp, dis, h1s, bg1, wg2)


def _g2_body(aggp_ref, dis_ref, h2s_ref, bg2_ref, x2_ref):
    dis = dis_ref[...]
    x2_ref[...] = dis * (aggp_ref[0] + aggp_ref[1] + h2s_ref[...]) + bg2_ref[...]


def _g2(aggp, dis, h2s, bg2):
    grid = NPAD // _RG
    return pl.pallas_call(
        _g2_body,
        grid=(grid,),
        in_specs=[
            pl.BlockSpec((NC, _RG, OUT), lambda i: (0, i, 0)),
            pl.BlockSpec((_RG, 1), lambda i: (i, 0)),
            pl.BlockSpec((_RG, OUT), lambda i: (i, 0)),
            pl.BlockSpec((1, OUT), lambda i: (0, 0)),
        ],
        out_specs=pl.BlockSpec((_RG, OUT), lambda i: (i, 0)),
        out_shape=jax.ShapeDtypeStruct((NPAD, OUT), jnp.float32),
    )(aggp, dis, h2s, bg2)


_RG3 = 2112


def _g3_body(ssump_ref, cntp_ref, x1s_ref, wa_ref, wb_ref, bls_ref, out_ref):
    cnt = jnp.maximum(cntp_ref[:, 0:1] + cntp_ref[:, 1:2], 1.0)
    x2n = (ssump_ref[0] + ssump_ref[1]) / cnt
    pred = (jnp.dot(x1s_ref[...], wa_ref[...], preferred_element_type=jnp.float32)
            + jnp.dot(x2n, wb_ref[...], preferred_element_type=jnp.float32)
            + bls_ref[...])
    out_ref[...] = jnp.maximum(pred, 0.0)


def _g3(ssump, cntpT, x1s, wa, wb, bls2):
    grid = GPAD // _RG3
    return pl.pallas_call(
        _g3_body,
        grid=(grid,),
        in_specs=[
            pl.BlockSpec((NC, _RG3, OUT), lambda i: (0, i, 0)),
            pl.BlockSpec((_RG3, NC), lambda i: (i, 0)),
            pl.BlockSpec((_RG3, OUT), lambda i: (i, 0)),
            pl.BlockSpec((OUT, 1), lambda i: (0, 0)),
            pl.BlockSpec((OUT, 1), lambda i: (0, 0)),
            pl.BlockSpec((1, 1), lambda i: (0, 0)),
        ],
        out_specs=pl.BlockSpec((_RG3, 1), lambda i: (i, 0)),
        out_shape=jax.ShapeDtypeStruct((GPAD, 1), jnp.float32),
    )(ssump, cntpT, x1s, wa, wb, bls2)


_PCH = 6400


def _s0_body(src_ref, dst_ref, w_ref, srco_ref, dsto_ref, wo_ref, bi, bf):
    core = lax.axis_index("c")
    sub = lax.axis_index("s")
    wid = sub * NC + core

    def chunk(ci, carry):
        base = wid * EPT + ci * _PCH
        for a, o, b in ((src_ref, srco_ref, bi), (dst_ref, dsto_ref, bi),
                        (w_ref, wo_ref, bf)):
            pltpu.sync_copy(a.at[pl.ds(base, _PCH)], b)
            pltpu.sync_copy(b, o.at[pl.ds(base, _PCH)])
        return carry
    lax.fori_loop(0, EPT // _PCH, chunk, 0)


@functools.cache
def _build_s0():
    return pl.kernel(
        _s0_body,
        out_type=(jax.ShapeDtypeStruct((E_PAD,), jnp.int32),
                  jax.ShapeDtypeStruct((E_PAD,), jnp.int32),
                  jax.ShapeDtypeStruct((E_PAD,), jnp.float32)),
        mesh=_mesh(),
        compiler_params=pltpu.CompilerParams(use_tc_tiling_on_sc=False),
        scratch_types=[pltpu.VMEM((_PCH,), jnp.int32),
                       pltpu.VMEM((_PCH,), jnp.float32)])


def _s0(*args):
    return _build_s0()(*args)


def _memset(ref, n, val):
    def step(i, carry):
        ref[pl.ds(i * 16, 16)] = jnp.full((16,), val, jnp.float32)
        return carry
    lax.fori_loop(0, n // 16, step, 0)


def _s1_body(dst_ref, w_ref, batch_ref, degout_ref, cntout_ref,
             dstv, wv, batchv, zbuf, onesv, dacc, cacc):
    core = lax.axis_index("c")
    sub = lax.axis_index("s")
    wid = sub * NC + core
    _memset(zbuf, NSTR, 0.0)
    _memset(onesv, 128, 1.0)
    pltpu.sync_copy(zbuf.at[pl.ds(0, NSTR)], dacc.at[pl.ds(sub * NSTR, NSTR)])
    pltpu.sync_copy(zbuf.at[pl.ds(0, GSTR)], cacc.at[pl.ds(sub * GSTR, GSTR)])
    plsc.subcore_barrier()

    def echunk(ci, carry):
        row0 = wid * (EPT // 128) + ci * (S1_CH // 128)
        pltpu.sync_copy(dst_ref.at[pl.ds(row0, S1_CH // 128)], dstv)
        pltpu.sync_copy(w_ref.at[pl.ds(row0, S1_CH // 128)], wv)
        for j in range(S1_CH // 128):
            pltpu.sync_copy(wv.at[j], dacc.at[dstv.at[j]], add=True)
        return carry
    lax.fori_loop(0, EPT // S1_CH, echunk, 0)

    for j in range(PPAD // NW // 128):
        pltpu.sync_copy(batch_ref.at[pl.ds(wid * (PPAD // NW) + j * 128, 128)],
                        batchv.at[j])
    for j in range(PPAD // NW // 128):
        pltpu.sync_copy(onesv, cacc.at[batchv.at[j]], add=True)
    plsc.subcore_barrier()

    pltpu.sync_copy(dacc.at[pl.ds(sub * NSTR, NSTR)], zbuf.at[pl.ds(0, NSTR)])
    pltpu.sync_copy(zbuf.at[pl.ds(0, NSTR)],
                    degout_ref.at[pl.ds(core * NPAD + sub * NSTR, NSTR)])
    pltpu.sync_copy(cacc.at[pl.ds(sub * GSTR, GSTR)], zbuf.at[pl.ds(0, GSTR)])
    pltpu.sync_copy(zbuf.at[pl.ds(0, GSTR)],
                    cntout_ref.at[pl.ds(core * GPAD + sub * GSTR, GSTR)])


@functools.cache
def _build_s1():
    return pl.kernel(
        _s1_body,
        out_type=(jax.ShapeDtypeStruct((NC * NPAD,), jnp.float32),
                  jax.ShapeDtypeStruct((NC * GPAD,), jnp.float32)),
        mesh=_mesh(),
        compiler_params=pltpu.CompilerParams(use_tc_tiling_on_sc=False),
        scratch_types=[
            pltpu.VMEM((S1_CH // 128, 128), jnp.int32),
            pltpu.VMEM((S1_CH // 128, 128), jnp.float32),
            pltpu.VMEM((PPAD // NW // 128, 128), jnp.int32),
            pltpu.VMEM((NSTR,), jnp.float32),
            pltpu.VMEM((128,), jnp.float32),
            pltpu.VMEM_SHARED((NPAD,), jnp.float32),
            pltpu.VMEM_SHARED((GPAD,), jnp.float32),
        ])


def _s1(*args):
    return _build_s1()(*args)


_NJ = S2_CH // 128
_NCH = EPT // S2_CH


def _s2_body(table_ref, src_ref, dst_ref, w_ref, out_ref,
             srcv, dstv, wv, rows, frows, semr0, semr1, semi0, semi1, acc):
    core = lax.axis_index("c")
    sub = lax.axis_index("s")
    wid = sub * NC + core
    semr = (semr0, semr1)
    semi = (semi0, semi1)

    def zrow(i, carry):
        frows[i, pl.ds(0, 16)] = jnp.zeros((16,), jnp.float32)
        frows[i, pl.ds(16, 16)] = jnp.zeros((16,), jnp.float32)
        return carry
    lax.fori_loop(0, SZB, zrow, 0)
    for k in range(NSTR // SZB):
        pltpu.sync_copy(frows.at[pl.ds(0, SZB)],
                        acc.at[pl.ds(sub * NSTR + k * SZB, SZB)])
    plsc.subcore_barrier()

    def idx_descs(c, b):
        base = wid * EPT + c * S2_CH
        ds_ = []
        for j in range(_NJ):
            ds_.append(pltpu.make_async_copy(
                src_ref.at[pl.ds(base + j * 128, 128)],
                srcv.at[_NJ * b + j], semi[b]))
            ds_.append(pltpu.make_async_copy(
                dst_ref.at[pl.ds(base + j * 128, 128)],
                dstv.at[_NJ * b + j], semi[b]))
        ds_.append(pltpu.make_async_copy(
            w_ref.at[pl.ds(base, S2_CH)], wv.at[b], semi[b]))
        return ds_

    def gather_descs(c, b):
        return [pltpu.make_async_copy(table_ref.at[srcv.at[_NJ * b + j]],
                                      rows.at[b, pl.ds(j * 128, 128)],
                                      semr[b])
                for j in range(_NJ)]

    def fire(descs):
        for d_ in descs:
            d_.start()

    def drain(descs):
        for d_ in descs:
            d_.wait()

    def process(c, b, fire_gnext, fire_inext):
        if fire_gnext:
            drain(idx_descs(c + 1, 1 - b))
            fire(gather_descs(c + 1, 1 - b))
        drain(gather_descs(c, b))
        for j in range(_NJ):
            def mul(i, c2, j=j):
                w16 = wv[b, pl.ds(j * 128 + i * 16, 16)]
                for l in range(16):
                    r = j * 128 + i * 16 + l
                    u = rows[b, r, pl.ds(0, 16)]
                    fa = plsc.bitcast(lax.shift_left(u, 16), jnp.float32)
                    fb = plsc.bitcast(u & jnp.int32(-65536), jnp.float32)
                    frows[i * 16 + l, pl.ds(0, 16)] = fa * w16[l]
                    frows[i * 16 + l, pl.ds(16, 16)] = fb * w16[l]
                return c2
            lax.fori_loop(0, 8, mul, 0)
            pltpu.sync_copy(frows.at[pl.ds(0, 128)],
                            acc.at[dstv.at[_NJ * b + j]], add=True)
        if fire_inext:
            fire(idx_descs(c + 2, b))

    fire(idx_descs(0, 0))
    drain(idx_descs(0, 0))
    fire(gather_descs(0, 0))
    fire(idx_descs(1, 1))

    def body(cj, carry):
        c = cj * 2
        process(c, 0, True, True)
        process(c + 1, 1, True, True)
        return carry
    lax.fori_loop(0, _NCH // 2 - 1, body, 0)
    process(_NCH - 2, 0, True, False)
    process(_NCH - 1, 1, False, False)
    plsc.subcore_barrier()

    for k in range(NSTR // SZB):
        pltpu.sync_copy(acc.at[pl.ds(sub * NSTR + k * SZB, SZB)],
                        frows.at[pl.ds(0, SZB)])
        pltpu.sync_copy(frows.at[pl.ds(0, SZB)],
                        out_ref.at[core, pl.ds(sub * NSTR + k * SZB, SZB)])


@functools.cache
def _build_s2():
    return pl.kernel(
        _s2_body,
        out_type=jax.ShapeDtypeStruct((NC, NPAD, HID), jnp.float32),
        mesh=_mesh(),
        compiler_params=pltpu.CompilerParams(use_tc_tiling_on_sc=False,
                                             needs_layout_passes=False),
        scratch_types=[
            pltpu.VMEM((2 * _NJ, 128), jnp.int32),
            pltpu.VMEM((2 * _NJ, 128), jnp.int32),
            pltpu.VMEM((2, S2_CH), jnp.float32),
            pltpu.VMEM((2, S2_CH, HID // 2), jnp.int32),
            pltpu.VMEM((SZB, HID), jnp.float32),
            pltpu.SemaphoreType.DMA,
            pltpu.SemaphoreType.DMA,
            pltpu.SemaphoreType.DMA,
            pltpu.SemaphoreType.DMA,
            pltpu.VMEM_SHARED((NPAD, HID), jnp.float32),
        ])


def _s2(*args):
    return _build_s2()(*args)


_PPT = PPAD // NW


def _s3_body(x2_ref, batch_ref, out_ref, batchv, rows, acc):
    core = lax.axis_index("c")
    sub = lax.axis_index("s")
    wid = sub * NC + core

    def zrow(i, carry):
        rows[i, pl.ds(0, 16)] = jnp.zeros((16,), jnp.float32)
        rows[i, pl.ds(16, 16)] = jnp.zeros((16,), jnp.float32)
        return carry
    lax.fori_loop(0, GSTR, zrow, 0)
    pltpu.sync_copy(rows.at[pl.ds(0, GSTR)], acc.at[pl.ds(sub * GSTR, GSTR)])
    plsc.subcore_barrier()

    pltpu.sync_copy(x2_ref.at[pl.ds(wid * _PPT, _PPT)], rows)
    for j in range(_PPT // 128):
        pltpu.sync_copy(batch_ref.at[pl.ds(wid * _PPT + j * 128, 128)],
                        batchv.at[j])
    for j in range(_PPT // 128):
        pltpu.sync_copy(rows.at[pl.ds(j * 128, 128)],
                        acc.at[batchv.at[j]], add=True)
    plsc.subcore_barrier()

    pltpu.sync_copy(acc.at[pl.ds(sub * GSTR, GSTR)], rows.at[pl.ds(0, GSTR)])
    pltpu.sync_copy(rows.at[pl.ds(0, GSTR)],
                    out_ref.at[core, pl.ds(sub * GSTR, GSTR)])


@functools.cache
def _build_s3():
    return pl.kernel(
        _s3_body,
        out_type=jax.ShapeDtypeStruct((NC, GPAD, OUT), jnp.float32),
        mesh=_mesh(),
        compiler_params=pltpu.CompilerParams(use_tc_tiling_on_sc=False),
        scratch_types=[
            pltpu.VMEM((_PPT // 128, 128), jnp.int32),
            pltpu.VMEM((_PPT, OUT), jnp.float32),
            pltpu.VMEM_SHARED((GPAD, OUT), jnp.float32),
        ])


def _s3(*args):
    return _build_s3()(*args)


def kernel(x, edge_index, edge_attr, batch, Wih_s, Whh_s, bih_s, bhh_s,
           Watt_s, Wfc_s, bfc_s, Wih_n, Whh_n, bih_n, bhh_n, Watt_n, Wfc_n,
           bfc_n, Wg1, bg1, Wg2, bg2, Wls, bls):
    f32 = jnp.float32
    x = x.astype(f32)

    xT_node = jnp.concatenate(
        [x.T, jnp.zeros((LAG, NPAD - N), f32)], 1).reshape(LAG, 1, NPAD)
    xsT = x.reshape(NG, 3, LAG).transpose(2, 1, 0)
    xsT = jnp.concatenate([xsT, jnp.zeros((LAG, 3, GPAD - NG), f32)], 2)
    src = edge_index[0].astype(jnp.int32)
    dst = edge_index[1].astype(jnp.int32)
    epad = E_PAD - E
    src_p = jnp.concatenate([src, jnp.zeros((epad,), jnp.int32)])
    dst_p = jnp.concatenate([dst, jnp.zeros((epad,), jnp.int32)])
    dst2 = dst_p.reshape(-1, 128)
    w_p = jnp.concatenate([edge_attr.astype(f32), jnp.zeros((epad,), f32)])
    batch1 = jnp.concatenate(
        [batch.astype(jnp.int32), jnp.full((PPAD - N,), NG, jnp.int32)])

    b_n = (bih_n + bhh_n).reshape(4 * HID, 1)
    b_s = (bih_s + bhh_s).reshape(4 * HID, 1)

    emb = _attlstm(xT_node, Wih_n, Whh_n, b_n,
                   Watt_n, Wfc_n, bfc_n.reshape(OUT, 1), R_LSTM=1024)

    src_e, dst_e, w_e = _s0(src_p, dst_p, w_p)
    degp, cntp = _s1(dst2, w_p.reshape(-1, 128), batch1)
    degp = degp.reshape(NC, NPAD)
    cntp = cntp.reshape(NC, GPAD)

    dis, h1s, t1 = _g0(degp.T, emb, Wg1)
    agg1p = _s2(t1, src_e, dst_e, w_e)

    xsT_b, _ = lax.optimization_barrier((xsT, h1s))
    x1s = _attlstm(xsT_b, Wih_s, Whh_s, b_s,
                   Watt_s, Wfc_s, bfc_s.reshape(OUT, 1))

    h2s, t2 = _g1(agg1p, dis, h1s, bg1.reshape(1, HID), Wg2)

    agg2p = _s2(t2, src_e, dst_e, w_e)
    x2 = _g2(agg2p, dis, h2s, bg2.reshape(1, OUT))

    x2_pool = jnp.concatenate([x2, jnp.zeros((PPAD - NPAD, OUT), f32)], 0)
    ssump = _s3(x2_pool, batch1)
    pred = _g3(ssump, cntp.T, x1s, Wls[:OUT].astype(f32),
               Wls[OUT:].astype(f32), bls.reshape(1, 1))
    return pred[:NG, 0]
